# Initial kernel scaffold; baseline (speedup 1.0000x reference)
import jax
import jax.numpy as jnp
from jax.experimental import pallas as pl


def kernel(input, edge_index, edge_relation, Ws0, bs0, Wl0, bl0, Ws1, bs1, Wl1, bl1):
    raise NotImplementedError("write your pallas kernel here")


# speedup vs baseline: 8.4210x; 8.4210x over previous
import functools

import jax
import jax.numpy as jnp
from jax import lax
from jax.experimental import pallas as pl
from jax.experimental.pallas import tpu as pltpu
from jax.experimental.pallas import tpu_sc as plsc

N = 10000
NP = 10240
E = 320000
D = 128
R = 7

NS = 16
NW = NS
EPW = E // NW
CH = 80
NCHUNK = EPW // CH
NBUF = 4
LA = 3
NBATCH = (NCHUNK + NBUF - 1) // NBUF

BN = 1000


def _transform1(x, Wt, b2):
    def body(x_ref, w_ref, b_ref, o_ref):
        o_ref[...] = jnp.dot(x_ref[...], w_ref[...],
                             preferred_element_type=jnp.float32) + b_ref[...]
    return pl.pallas_call(
        body,
        grid=(N // BN,),
        in_specs=[
            pl.BlockSpec((BN, D), lambda i: (i, 0)),
            pl.BlockSpec((D, D * R), lambda i: (0, 0)),
            pl.BlockSpec((1, D * R), lambda i: (0, 0)),
        ],
        out_specs=pl.BlockSpec((BN, D * R), lambda i: (i, 0)),
        out_shape=jax.ShapeDtypeStruct((N, D * R), jnp.float32),
    )(x, Wt, b2)


def _transform2(p, Wt, b2):
    def body(p_ref, w_ref, b_ref, o_ref):
        x = jnp.maximum(p_ref[...], 0.0)
        o_ref[...] = jnp.dot(x, w_ref[...],
                             preferred_element_type=jnp.float32) + b_ref[...]
    return pl.pallas_call(
        body,
        grid=(N // BN,),
        in_specs=[
            pl.BlockSpec((BN, D), lambda i: (i, 0)),
            pl.BlockSpec((D, D * R), lambda i: (0, 0)),
            pl.BlockSpec((1, D * R), lambda i: (0, 0)),
        ],
        out_specs=pl.BlockSpec((BN, D * R), lambda i: (i, 0)),
        out_shape=jax.ShapeDtypeStruct((N, D * R), jnp.float32),
    )(p, Wt, b2)


def _finalize(p):
    def body(p_ref, g_ref, n_ref):
        i = pl.program_id(0)
        nb = jnp.maximum(p_ref[...], 0.0)
        n_ref[...] = nb

        @pl.when(i == 0)
        def _():
            g_ref[...] = jnp.zeros_like(g_ref)

        g_ref[...] += jnp.sum(nb, axis=0, keepdims=True)

    graph, node = pl.pallas_call(
        body,
        grid=(N // BN,),
        in_specs=[pl.BlockSpec((BN, D), lambda i: (i, 0))],
        out_specs=[pl.BlockSpec((1, D), lambda i: (0, 0)),
                   pl.BlockSpec((BN, D), lambda i: (i, 0))],
        out_shape=[jax.ShapeDtypeStruct((1, D), jnp.float32),
                   jax.ShapeDtypeStruct((N, D), jnp.float32)],
    )(p)
    return graph.reshape(D), node


def _edge_scatter(src, rel, dst, z, zero):
    mesh = plsc.VectorSubcoreMesh(core_axis_name="c", subcore_axis_name="s",
                                  num_cores=1)

    @functools.partial(
        pl.kernel,
        mesh=mesh,
        out_type=jax.ShapeDtypeStruct((NP, D), jnp.float32),
        scratch_types=[
            pltpu.VMEM((NBUF, CH), jnp.int32),
            pltpu.VMEM((NBUF, CH), jnp.int32),
            pltpu.VMEM((NBUF, CH), jnp.int32),
            pltpu.VMEM((NBUF, CH, D), jnp.float32),
            pltpu.VMEM_SHARED((NP, D), jnp.float32),
        ] + [pltpu.SemaphoreType.DMA] * (2 * NBUF),
    )
    def body(src_h, rel_h, dst_h, z_h, zero_h, out_h,
             sbuf, rbuf, dbuf, rows_v, acc, *sems):
        sem_i = sems[:NBUF]
        sem_g = sems[NBUF:]
        sid = lax.axis_index("s")
        rpt = NP // NS
        r0 = sid * rpt
        e0 = sid * EPW

        pltpu.sync_copy(zero_h.at[pl.ds(r0, rpt)], acc.at[pl.ds(r0, rpt)])
        plsc.subcore_barrier()

        def fire_idx(k, b):
            base = pl.multiple_of(e0 + k * CH, 8)
            pltpu.async_copy(src_h.at[pl.ds(base, CH)], sbuf.at[b], sem_i[b])
            pltpu.async_copy(rel_h.at[pl.ds(base, CH)], rbuf.at[b], sem_i[b])
            pltpu.async_copy(dst_h.at[pl.ds(base, CH)], dbuf.at[b], sem_i[b])

        def wait_idx(b):
            pltpu.make_async_copy(src_h.at[pl.ds(0, CH)], sbuf.at[b],
                                  sem_i[b]).wait()
            pltpu.make_async_copy(src_h.at[pl.ds(0, CH)], rbuf.at[b],
                                  sem_i[b]).wait()
            pltpu.make_async_copy(src_h.at[pl.ds(0, CH)], dbuf.at[b],
                                  sem_i[b]).wait()

        def fire_gather(b):
            for j in range(CH // 16):
                sl = pl.ds(j * 16, 16)
                sbuf[b, sl] = sbuf[b, sl] * R + rbuf[b, sl]
            pltpu.async_copy(z_h.at[sbuf.at[b]], rows_v.at[b], sem_g[b])

        def wait_gather(b):
            pltpu.make_async_copy(z_h.at[pl.ds(0, CH)], rows_v.at[b],
                                  sem_g[b]).wait()

        for b in range(NBUF):
            fire_idx(b, b)
        for b in range(LA):
            wait_idx(b)
            fire_gather(b)

        def batch(i, c):
            for b in range(NBUF):
                k = i * NBUF + b
                kk = k + LA
                ba = (b + LA) % NBUF

                @pl.when(kk < NCHUNK)
                def _():
                    wait_idx(ba)
                    fire_gather(ba)

                @pl.when(k < NCHUNK)
                def _():
                    wait_gather(b)
                    pltpu.sync_copy(rows_v.at[b], acc.at[dbuf.at[b]],
                                    add=True)
                kn = k + NBUF

                @pl.when(kn < NCHUNK)
                def _():
                    fire_idx(kn, b)
            return c
        lax.fori_loop(0, NBATCH, batch, 0)

        plsc.subcore_barrier()
        pltpu.sync_copy(acc.at[pl.ds(r0, rpt)], out_h.at[pl.ds(r0, rpt)])

    return body(src, rel, dst, z, zero)


def kernel(input, edge_index, edge_relation, Ws0, bs0, Wl0, bl0,
           Ws1, bs1, Wl1, bl1):
    x = input
    src = edge_index[0]
    dst = edge_index[1]
    rel = edge_relation
    zero = jnp.zeros((NP, D), jnp.float32)

    z1 = _transform1(x, Wl0.T, bl0.reshape(1, D * R)).reshape(N * R, D)
    p1 = _edge_scatter(src, rel, dst, z1, zero)
    z2 = _transform2(p1, Wl1.T, bl1.reshape(1, D * R)).reshape(N * R, D)
    p2 = _edge_scatter(src, rel, dst, z2, zero)
    return _finalize(p2)


# speedup vs baseline: 12.4337x; 1.4765x over previous
import functools

import jax
import jax.numpy as jnp
from jax import lax
from jax.experimental import pallas as pl
from jax.experimental.pallas import tpu as pltpu
from jax.experimental.pallas import tpu_sc as plsc

N = 10000
E = 320000
D = 128
R = 7

NC = 2
NS = 16
HALF = N // 2
NPH = 5120
EPS = E // NS
SCH = 2000
NMEGA = EPS // SCH
CAP = 20160
CH = 80
CAPCH = CAP // CH
NBUF = 2
LA = 1
NBATCH = (CAPCH + NBUF - 1) // NBUF

BN = 1000


def _transform1(x, Wt, b2):
    def body(x_ref, w_ref, b_ref, o_ref):
        o_ref[...] = jnp.dot(x_ref[...], w_ref[...],
                             preferred_element_type=jnp.float32) + b_ref[...]
    return pl.pallas_call(
        body,
        grid=(N // BN,),
        in_specs=[
            pl.BlockSpec((BN, D), lambda i: (i, 0)),
            pl.BlockSpec((D, D * R), lambda i: (0, 0)),
            pl.BlockSpec((1, D * R), lambda i: (0, 0)),
        ],
        out_specs=pl.BlockSpec((BN, D * R), lambda i: (i, 0)),
        out_shape=jax.ShapeDtypeStruct((N, D * R), jnp.float32),
    )(x, Wt, b2)


def _transform2(p, Wt, b2):
    nb = N // BN // NC

    def body(p_ref, w_ref, b_ref, o_ref):
        x = jnp.maximum(p_ref[0], 0.0)
        o_ref[...] = jnp.dot(x, w_ref[...],
                             preferred_element_type=jnp.float32) + b_ref[...]
    return pl.pallas_call(
        body,
        grid=(N // BN,),
        in_specs=[
            pl.BlockSpec((1, BN, D), lambda i: (i // nb, i % nb, 0)),
            pl.BlockSpec((D, D * R), lambda i: (0, 0)),
            pl.BlockSpec((1, D * R), lambda i: (0, 0)),
        ],
        out_specs=pl.BlockSpec((BN, D * R), lambda i: (i, 0)),
        out_shape=jax.ShapeDtypeStruct((N, D * R), jnp.float32),
    )(p, Wt, b2)


def _finalize(p):
    nb = N // BN // NC

    def body(p_ref, g_ref, n_ref):
        i = pl.program_id(0)
        x = jnp.maximum(p_ref[0], 0.0)
        n_ref[...] = x

        @pl.when(i == 0)
        def _():
            g_ref[...] = jnp.zeros_like(g_ref)

        g_ref[...] += jnp.sum(x, axis=0, keepdims=True)

    graph, node = pl.pallas_call(
        body,
        grid=(N // BN,),
        in_specs=[pl.BlockSpec((1, BN, D), lambda i: (i // nb, i % nb, 0))],
        out_specs=[pl.BlockSpec((1, D), lambda i: (0, 0)),
                   pl.BlockSpec((BN, D), lambda i: (i, 0))],
        out_shape=[jax.ShapeDtypeStruct((1, D), jnp.float32),
                   jax.ShapeDtypeStruct((N, D), jnp.float32)],
    )(p)
    return graph.reshape(D), node


def _edge_prep(src, rel, dst):
    mesh = plsc.VectorSubcoreMesh(core_axis_name="c", subcore_axis_name="s")

    @functools.partial(
        pl.kernel,
        mesh=mesh,
        compiler_params=pltpu.CompilerParams(needs_layout_passes=False),
        out_type=(
            jax.ShapeDtypeStruct((NC * NS * CAP,), jnp.int32),
            jax.ShapeDtypeStruct((NC * NS * CAP,), jnp.int32),
            jax.ShapeDtypeStruct((NC * NS * 16,), jnp.int32),
        ),
        scratch_types=[
            pltpu.VMEM((SCH,), jnp.int32),
            pltpu.VMEM((SCH,), jnp.int32),
            pltpu.VMEM((SCH,), jnp.int32),
            pltpu.VMEM((SCH,), jnp.int32),
            pltpu.VMEM((SCH,), jnp.int32),
            pltpu.VMEM((SCH,), jnp.int32),
            pltpu.VMEM((CAP,), jnp.int32),
            pltpu.VMEM((CAP,), jnp.int32),
            pltpu.VMEM((16,), jnp.int32),
            pltpu.SemaphoreType.DMA,
            pltpu.SemaphoreType.DMA,
        ],
    )
    def body(src_h, rel_h, dst_h, cg_h, cd_h, cnt_h,
             src_s0, src_s1, rel_s0, rel_s1, dst_s0, dst_s1,
             ccg, ccd, cbuf, sem0, sem1):
        src_s = (src_s0, src_s1)
        rel_s = (rel_s0, rel_s1)
        dst_s = (dst_s0, dst_s1)
        cid = lax.axis_index("c")
        sid = lax.axis_index("s")
        sems = (sem0, sem1)
        e0 = sid * EPS
        lo = cid * HALF

        def fire(m, b):
            base = pl.multiple_of(e0 + m * SCH, 8)
            pltpu.async_copy(src_h.at[pl.ds(base, SCH)], src_s[b], sems[b])
            pltpu.async_copy(rel_h.at[pl.ds(base, SCH)], rel_s[b], sems[b])
            pltpu.async_copy(dst_h.at[pl.ds(base, SCH)], dst_s[b], sems[b])

        def wait(b):
            for ref in (src_s, rel_s, dst_s):
                pltpu.make_async_copy(src_h.at[pl.ds(0, SCH)], ref[b],
                                      sems[b]).wait()

        fire(0, 0)
        iota = lax.iota(jnp.int32, 16)

        def pre(i, c):
            sl = pl.ds(i * 16, 16)
            ccg[sl] = (iota + i * 16) & 8191
            ccd[sl] = HALF + ((iota + i) & 63)
            return c
        lax.fori_loop(0, CAP // 16, pre, 0)

        def mega(i, fillv):
            for b in range(2):
                m = i * 2 + b
                wait(b)

                @pl.when(m + 1 < NMEGA)
                def _():
                    fire(m + 1, 1 - b)

                def scan(v, fv):
                    sl = pl.ds(v * 16, 16)
                    s = src_s[b][sl]
                    r = rel_s[b][sl]
                    d = dst_s[b][sl] - lo
                    g = s * R + r
                    keep = (d >= 0) & (d < HALF)
                    ki = keep.astype(jnp.int32)
                    pos = fv * 16 + iota
                    plsc.store_scatter(ccg, [pos], g, mask=keep)
                    plsc.store_scatter(ccd, [pos], d, mask=keep)
                    return fv + ki
                fillv = lax.fori_loop(0, SCH // 16, scan, fillv)
            return fillv
        fillv = lax.fori_loop(0, NMEGA // 2, mega, jnp.zeros(16, jnp.int32))

        mx = fillv[0]
        for l in range(1, 16):
            mx = jnp.maximum(mx, fillv[l])
        nchunks = (mx * 16 + CH - 1) // CH

        obase = (cid * NS + sid) * CAP
        pltpu.sync_copy(ccg, cg_h.at[pl.ds(obase, CAP)])
        pltpu.sync_copy(ccd, cd_h.at[pl.ds(obase, CAP)])
        cbuf[...] = jnp.full((16,), nchunks, jnp.int32)
        pltpu.sync_copy(cbuf, cnt_h.at[pl.ds((cid * NS + sid) * 16, 16)])

    return body(src, rel, dst)


def _edge_scatter(cg, cd, cnt, z, zero):
    mesh = plsc.VectorSubcoreMesh(core_axis_name="c", subcore_axis_name="s")

    @functools.partial(
        pl.kernel,
        mesh=mesh,
        compiler_params=pltpu.CompilerParams(needs_layout_passes=False),
        out_type=jax.ShapeDtypeStruct((NC, NPH, D), jnp.float32),
        scratch_types=[
            pltpu.VMEM((NBUF, CH), jnp.int32),
            pltpu.VMEM((NBUF, CH), jnp.int32),
            pltpu.VMEM((NBUF, CH, D), jnp.float32),
            pltpu.VMEM((16,), jnp.int32),
            pltpu.VMEM_SHARED((NPH, D), jnp.float32),
        ] + [pltpu.SemaphoreType.DMA] * (2 * NBUF + 1),
    )
    def body(cg_h, cd_h, cnt_h, z_h, zero_h, out_h,
             gbuf, dbuf, rows_v, cbuf, acc, *sems):
        sem_i = sems[:NBUF]
        sem_g = sems[NBUF:2 * NBUF]
        sem_c = sems[2 * NBUF]
        cid = lax.axis_index("c")
        sid = lax.axis_index("s")
        rpt = NPH // NS
        r0 = sid * rpt
        obase = (cid * NS + sid) * CAP

        pltpu.async_copy(cnt_h.at[pl.ds((cid * NS + sid) * 16, 16)], cbuf,
                         sem_c)
        pltpu.sync_copy(zero_h.at[pl.ds(r0, rpt)], acc.at[pl.ds(r0, rpt)])
        pltpu.make_async_copy(cnt_h.at[pl.ds(0, 16)], cbuf, sem_c).wait()
        nchunks = cbuf[pl.ds(0, 16)][0]
        plsc.subcore_barrier()

        def fire_idx(k, b):
            base = pl.multiple_of(obase + k * CH, 8)
            pltpu.async_copy(cg_h.at[pl.ds(base, CH)], gbuf.at[b], sem_i[b])
            pltpu.async_copy(cd_h.at[pl.ds(base, CH)], dbuf.at[b], sem_i[b])

        def wait_idx(b):
            pltpu.make_async_copy(cg_h.at[pl.ds(0, CH)], gbuf.at[b],
                                  sem_i[b]).wait()
            pltpu.make_async_copy(cg_h.at[pl.ds(0, CH)], dbuf.at[b],
                                  sem_i[b]).wait()

        def fire_gather(b):
            pltpu.async_copy(z_h.at[gbuf.at[b]], rows_v.at[b], sem_g[b])

        def wait_gather(b):
            pltpu.make_async_copy(z_h.at[pl.ds(0, CH)], rows_v.at[b],
                                  sem_g[b]).wait()

        for b in range(NBUF):
            @pl.when(b < nchunks)
            def _():
                fire_idx(b, b)
        for b in range(LA):
            @pl.when(b < nchunks)
            def _():
                wait_idx(b)
                fire_gather(b)

        def batch(i, c):
            for b in range(NBUF):
                k = i * NBUF + b
                kk = k + LA
                ba = (b + LA) % NBUF

                @pl.when(kk < nchunks)
                def _():
                    wait_idx(ba)
                    fire_gather(ba)

                @pl.when(k < nchunks)
                def _():
                    wait_gather(b)
                    pltpu.sync_copy(rows_v.at[b], acc.at[dbuf.at[b]],
                                    add=True)
                kn = k + NBUF

                @pl.when(kn < nchunks)
                def _():
                    fire_idx(kn, b)
            return c
        lax.fori_loop(0, NBATCH, batch, 0)

        plsc.subcore_barrier()
        pltpu.sync_copy(acc.at[pl.ds(r0, rpt)], out_h.at[cid, pl.ds(r0, rpt)])

    return body(cg, cd, cnt, z, zero)


def kernel(input, edge_index, edge_relation, Ws0, bs0, Wl0, bl0,
           Ws1, bs1, Wl1, bl1):
    x = input
    src = edge_index[0]
    dst = edge_index[1]
    rel = edge_relation
    zero = jnp.zeros((NPH, D), jnp.float32)

    cg, cd, cnt = _edge_prep(src, rel, dst)
    z1 = _transform1(x, Wl0.T, bl0.reshape(1, D * R)).reshape(N * R, D)
    p1 = _edge_scatter(cg, cd, cnt, z1, zero)
    z2 = _transform2(p1, Wl1.T, bl1.reshape(1, D * R)).reshape(N * R, D)
    p2 = _edge_scatter(cg, cd, cnt, z2, zero)
    return _finalize(p2)


# speedup vs baseline: 14.3030x; 1.1503x over previous
import functools

import jax
import jax.numpy as jnp
from jax import lax
from jax.experimental import pallas as pl
from jax.experimental.pallas import tpu as pltpu
from jax.experimental.pallas import tpu_sc as plsc

N = 10000
E = 320000
D = 128
R = 7

NC = 2
NS = 16
HALF = N // 2
NPH = 5120
EPS = E // NS
SCH = 2000
NMEGA = EPS // SCH
CAP = 20160
CH = 80
CAPCH = CAP // CH
NIB = 4
NRB = 2
NBATCH = 63

BN = 1000


def _transform1(x, Wt, b2):
    def body(x_ref, w_ref, b_ref, o_ref):
        o_ref[...] = jnp.dot(x_ref[...], w_ref[...],
                             preferred_element_type=jnp.float32) + b_ref[...]
    return pl.pallas_call(
        body,
        grid=(N // BN,),
        in_specs=[
            pl.BlockSpec((BN, D), lambda i: (i, 0)),
            pl.BlockSpec((D, D * R), lambda i: (0, 0)),
            pl.BlockSpec((1, D * R), lambda i: (0, 0)),
        ],
        out_specs=pl.BlockSpec((BN, D * R), lambda i: (i, 0)),
        out_shape=jax.ShapeDtypeStruct((N, D * R), jnp.float32),
    )(x, Wt, b2)


def _transform2(p, Wt, b2):
    nb = N // BN // NC

    def body(p_ref, w_ref, b_ref, o_ref):
        x = jnp.maximum(p_ref[0], 0.0)
        o_ref[...] = jnp.dot(x, w_ref[...],
                             preferred_element_type=jnp.float32) + b_ref[...]
    return pl.pallas_call(
        body,
        grid=(N // BN,),
        in_specs=[
            pl.BlockSpec((1, BN, D), lambda i: (i // nb, i % nb, 0)),
            pl.BlockSpec((D, D * R), lambda i: (0, 0)),
            pl.BlockSpec((1, D * R), lambda i: (0, 0)),
        ],
        out_specs=pl.BlockSpec((BN, D * R), lambda i: (i, 0)),
        out_shape=jax.ShapeDtypeStruct((N, D * R), jnp.float32),
    )(p, Wt, b2)


def _finalize(p):
    nb = N // BN // NC

    def body(p_ref, g_ref, n_ref):
        i = pl.program_id(0)
        x = jnp.maximum(p_ref[0], 0.0)
        n_ref[...] = x

        @pl.when(i == 0)
        def _():
            g_ref[...] = jnp.zeros_like(g_ref)

        g_ref[...] += jnp.sum(x, axis=0, keepdims=True)

    graph, node = pl.pallas_call(
        body,
        grid=(N // BN,),
        in_specs=[pl.BlockSpec((1, BN, D), lambda i: (i // nb, i % nb, 0))],
        out_specs=[pl.BlockSpec((1, D), lambda i: (0, 0)),
                   pl.BlockSpec((BN, D), lambda i: (i, 0))],
        out_shape=[jax.ShapeDtypeStruct((1, D), jnp.float32),
                   jax.ShapeDtypeStruct((N, D), jnp.float32)],
    )(p)
    return graph.reshape(D), node


def _edge_prep(src, rel, dst):
    mesh = plsc.VectorSubcoreMesh(core_axis_name="c", subcore_axis_name="s")

    @functools.partial(
        pl.kernel,
        mesh=mesh,
        compiler_params=pltpu.CompilerParams(needs_layout_passes=False),
        out_type=(
            jax.ShapeDtypeStruct((NC * NS * CAP,), jnp.int32),
            jax.ShapeDtypeStruct((NC * NS * CAP,), jnp.int32),
            jax.ShapeDtypeStruct((NC * NS * 16,), jnp.int32),
        ),
        scratch_types=[
            pltpu.VMEM((SCH,), jnp.int32),
            pltpu.VMEM((SCH,), jnp.int32),
            pltpu.VMEM((SCH,), jnp.int32),
            pltpu.VMEM((SCH,), jnp.int32),
            pltpu.VMEM((SCH,), jnp.int32),
            pltpu.VMEM((SCH,), jnp.int32),
            pltpu.VMEM((CAP,), jnp.int32),
            pltpu.VMEM((CAP,), jnp.int32),
            pltpu.VMEM((16,), jnp.int32),
            pltpu.SemaphoreType.DMA,
            pltpu.SemaphoreType.DMA,
        ],
    )
    def body(src_h, rel_h, dst_h, cg_h, cd_h, cnt_h,
             src_s0, src_s1, rel_s0, rel_s1, dst_s0, dst_s1,
             ccg, ccd, cbuf, sem0, sem1):
        src_s = (src_s0, src_s1)
        rel_s = (rel_s0, rel_s1)
        dst_s = (dst_s0, dst_s1)
        cid = lax.axis_index("c")
        sid = lax.axis_index("s")
        sems = (sem0, sem1)
        e0 = sid * EPS
        lo = cid * HALF

        def fire(m, b):
            base = pl.multiple_of(e0 + m * SCH, 8)
            pltpu.async_copy(src_h.at[pl.ds(base, SCH)], src_s[b], sems[b])
            pltpu.async_copy(rel_h.at[pl.ds(base, SCH)], rel_s[b], sems[b])
            pltpu.async_copy(dst_h.at[pl.ds(base, SCH)], dst_s[b], sems[b])

        def wait(b):
            for ref in (src_s, rel_s, dst_s):
                pltpu.make_async_copy(src_h.at[pl.ds(0, SCH)], ref[b],
                                      sems[b]).wait()

        fire(0, 0)
        iota = lax.iota(jnp.int32, 16)

        def pre(i, c):
            sl = pl.ds(i * 16, 16)
            ccg[sl] = (iota + i * 16) & 8191
            ccd[sl] = HALF + ((iota + i) & 63)
            return c
        lax.fori_loop(0, CAP // 16, pre, 0)

        def mega(i, fillv):
            for b in range(2):
                m = i * 2 + b
                wait(b)

                @pl.when(m + 1 < NMEGA)
                def _():
                    fire(m + 1, 1 - b)

                def scan(v, fv):
                    sl = pl.ds(v * 16, 16)
                    s = src_s[b][sl]
                    r = rel_s[b][sl]
                    d = dst_s[b][sl] - lo
                    g = s * R + r
                    keep = (d >= 0) & (d < HALF)
                    ki = keep.astype(jnp.int32)
                    pos = fv * 16 + iota
                    plsc.store_scatter(ccg, [pos], g, mask=keep)
                    plsc.store_scatter(ccd, [pos], d, mask=keep)
                    return fv + ki
                fillv = lax.fori_loop(0, SCH // 16, scan, fillv)
            return fillv
        fillv = lax.fori_loop(0, NMEGA // 2, mega, jnp.zeros(16, jnp.int32))

        mx = fillv[0]
        for l in range(1, 16):
            mx = jnp.maximum(mx, fillv[l])
        nchunks = (mx * 16 + CH - 1) // CH

        obase = (cid * NS + sid) * CAP
        pltpu.sync_copy(ccg, cg_h.at[pl.ds(obase, CAP)])
        pltpu.sync_copy(ccd, cd_h.at[pl.ds(obase, CAP)])
        cbuf[...] = jnp.full((16,), nchunks, jnp.int32)
        pltpu.sync_copy(cbuf, cnt_h.at[pl.ds((cid * NS + sid) * 16, 16)])

    return body(src, rel, dst)


def _edge_scatter(cg, cd, cnt, z, zero):
    mesh = plsc.VectorSubcoreMesh(core_axis_name="c", subcore_axis_name="s")

    @functools.partial(
        pl.kernel,
        mesh=mesh,
        compiler_params=pltpu.CompilerParams(needs_layout_passes=False),
        out_type=jax.ShapeDtypeStruct((NC, NPH, D), jnp.float32),
        scratch_types=[
            pltpu.VMEM((NIB, CH), jnp.int32),
            pltpu.VMEM((NIB, CH), jnp.int32),
            pltpu.VMEM((NRB, CH, D), jnp.float32),
            pltpu.VMEM((16,), jnp.int32),
            pltpu.VMEM_SHARED((NPH, D), jnp.float32),
        ] + [pltpu.SemaphoreType.DMA] * (NIB + 2 * NRB + 1),
    )
    def body(cg_h, cd_h, cnt_h, z_h, zero_h, out_h,
             gbuf, dbuf, rows_v, cbuf, acc, *sems):
        sem_i = sems[:NIB]
        sem_g = sems[NIB:NIB + NRB]
        sem_s = sems[NIB + NRB:NIB + 2 * NRB]
        sem_c = sems[NIB + 2 * NRB]
        cid = lax.axis_index("c")
        sid = lax.axis_index("s")
        rpt = NPH // NS
        r0 = sid * rpt
        obase = (cid * NS + sid) * CAP

        pltpu.async_copy(cnt_h.at[pl.ds((cid * NS + sid) * 16, 16)], cbuf,
                         sem_c)
        pltpu.sync_copy(zero_h.at[pl.ds(r0, rpt)], acc.at[pl.ds(r0, rpt)])
        pltpu.make_async_copy(cnt_h.at[pl.ds(0, 16)], cbuf, sem_c).wait()
        nchunks = cbuf[pl.ds(0, 16)][0]
        plsc.subcore_barrier()

        def fire_idx(k, ib):
            base = pl.multiple_of(obase + k * CH, 8)
            pltpu.async_copy(cg_h.at[pl.ds(base, CH)], gbuf.at[ib], sem_i[ib])
            pltpu.async_copy(cd_h.at[pl.ds(base, CH)], dbuf.at[ib], sem_i[ib])

        def wait_idx(ib):
            pltpu.make_async_copy(cg_h.at[pl.ds(0, CH)], gbuf.at[ib],
                                  sem_i[ib]).wait()
            pltpu.make_async_copy(cg_h.at[pl.ds(0, CH)], dbuf.at[ib],
                                  sem_i[ib]).wait()

        def fire_gather(ib, rb):
            pltpu.async_copy(z_h.at[gbuf.at[ib]], rows_v.at[rb], sem_g[rb])

        def wait_gather(rb):
            pltpu.make_async_copy(z_h.at[pl.ds(0, CH)], rows_v.at[rb],
                                  sem_g[rb]).wait()

        def fire_scatter(rb, ib):
            pltpu.async_copy(rows_v.at[rb], acc.at[dbuf.at[ib]], sem_s[rb],
                             add=True)

        def wait_scatter(rb):
            pltpu.make_async_copy(rows_v.at[rb], acc.at[pl.ds(0, CH)],
                                  sem_s[rb]).wait()

        for u in range(3):
            @pl.when(u < nchunks)
            def _():
                fire_idx(u, u)

        @pl.when(0 < nchunks)
        def _():
            wait_idx(0)
            fire_gather(0, 0)

        def batch(i, c):
            for u in range(4):
                k = i * 4 + u
                rb = u % 2
                ib = u

                @pl.when((k >= 1) & (k - 1 < nchunks))
                def _():
                    wait_scatter((u + 1) % 2)

                @pl.when(k + 3 < nchunks)
                def _():
                    fire_idx(k + 3, (u + 3) % 4)

                @pl.when(k + 1 < nchunks)
                def _():
                    wait_idx((u + 1) % 4)
                    fire_gather((u + 1) % 4, (u + 1) % 2)

                @pl.when(k < nchunks)
                def _():
                    wait_gather(rb)
                    fire_scatter(rb, ib)
            return c
        lax.fori_loop(0, NBATCH, batch, 0)

        plsc.subcore_barrier()
        pltpu.sync_copy(acc.at[pl.ds(r0, rpt)], out_h.at[cid, pl.ds(r0, rpt)])

    return body(cg, cd, cnt, z, zero)


def kernel(input, edge_index, edge_relation, Ws0, bs0, Wl0, bl0,
           Ws1, bs1, Wl1, bl1):
    x = input
    src = edge_index[0]
    dst = edge_index[1]
    rel = edge_relation
    zero = jnp.zeros((NPH, D), jnp.float32)

    cg, cd, cnt = _edge_prep(src, rel, dst)
    z1 = _transform1(x, Wl0.T, bl0.reshape(1, D * R)).reshape(N * R, D)
    p1 = _edge_scatter(cg, cd, cnt, z1, zero)
    z2 = _transform2(p1, Wl1.T, bl1.reshape(1, D * R)).reshape(N * R, D)
    p2 = _edge_scatter(cg, cd, cnt, z2, zero)
    return _finalize(p2)


# speedup vs baseline: 14.5117x; 1.0146x over previous
import functools

import jax
import jax.numpy as jnp
from jax import lax
from jax.experimental import pallas as pl
from jax.experimental.pallas import tpu as pltpu
from jax.experimental.pallas import tpu_sc as plsc

N = 10000
E = 320000
D = 128
R = 7

NC = 2
NS = 16
HALF = N // 2
NPH = 5120
EPS = E // NS
SCH = 2000
NMEGA = EPS // SCH
CAP = 20064
CH = 88
CAPCH = CAP // CH
NIB = 4
NRB = 2
NBATCH = 58

BN = 1000


def _transform1(x, Wt, b2):
    def body(x_ref, w_ref, b_ref, o_ref):
        o_ref[...] = jnp.dot(x_ref[...], w_ref[...],
                             preferred_element_type=jnp.float32) + b_ref[...]
    return pl.pallas_call(
        body,
        grid=(N // BN,),
        in_specs=[
            pl.BlockSpec((BN, D), lambda i: (i, 0)),
            pl.BlockSpec((D, D * R), lambda i: (0, 0)),
            pl.BlockSpec((1, D * R), lambda i: (0, 0)),
        ],
        out_specs=pl.BlockSpec((BN, D * R), lambda i: (i, 0)),
        out_shape=jax.ShapeDtypeStruct((N, D * R), jnp.float32),
    )(x, Wt, b2)


def _transform2(p, Wt, b2):
    nb = N // BN // NC

    def body(p_ref, w_ref, b_ref, o_ref):
        x = jnp.maximum(p_ref[0], 0.0)
        o_ref[...] = jnp.dot(x, w_ref[...],
                             preferred_element_type=jnp.float32) + b_ref[...]
    return pl.pallas_call(
        body,
        grid=(N // BN,),
        in_specs=[
            pl.BlockSpec((1, BN, D), lambda i: (i // nb, i % nb, 0)),
            pl.BlockSpec((D, D * R), lambda i: (0, 0)),
            pl.BlockSpec((1, D * R), lambda i: (0, 0)),
        ],
        out_specs=pl.BlockSpec((BN, D * R), lambda i: (i, 0)),
        out_shape=jax.ShapeDtypeStruct((N, D * R), jnp.float32),
    )(p, Wt, b2)


def _finalize(p):
    nb = N // BN // NC

    def body(p_ref, g_ref, n_ref):
        i = pl.program_id(0)
        x = jnp.maximum(p_ref[0], 0.0)
        n_ref[...] = x

        @pl.when(i == 0)
        def _():
            g_ref[...] = jnp.zeros_like(g_ref)

        g_ref[...] += jnp.sum(x, axis=0, keepdims=True)

    graph, node = pl.pallas_call(
        body,
        grid=(N // BN,),
        in_specs=[pl.BlockSpec((1, BN, D), lambda i: (i // nb, i % nb, 0))],
        out_specs=[pl.BlockSpec((1, D), lambda i: (0, 0)),
                   pl.BlockSpec((BN, D), lambda i: (i, 0))],
        out_shape=[jax.ShapeDtypeStruct((1, D), jnp.float32),
                   jax.ShapeDtypeStruct((N, D), jnp.float32)],
    )(p)
    return graph.reshape(D), node


def _edge_prep(src, rel, dst):
    mesh = plsc.VectorSubcoreMesh(core_axis_name="c", subcore_axis_name="s")

    @functools.partial(
        pl.kernel,
        mesh=mesh,
        compiler_params=pltpu.CompilerParams(needs_layout_passes=False),
        out_type=(
            jax.ShapeDtypeStruct((NC * NS * CAP,), jnp.int32),
            jax.ShapeDtypeStruct((NC * NS * CAP,), jnp.int32),
            jax.ShapeDtypeStruct((NC * NS * 16,), jnp.int32),
        ),
        scratch_types=[
            pltpu.VMEM((SCH,), jnp.int32),
            pltpu.VMEM((SCH,), jnp.int32),
            pltpu.VMEM((SCH,), jnp.int32),
            pltpu.VMEM((SCH,), jnp.int32),
            pltpu.VMEM((SCH,), jnp.int32),
            pltpu.VMEM((SCH,), jnp.int32),
            pltpu.VMEM((CAP,), jnp.int32),
            pltpu.VMEM((CAP,), jnp.int32),
            pltpu.VMEM((16,), jnp.int32),
            pltpu.SemaphoreType.DMA,
            pltpu.SemaphoreType.DMA,
        ],
    )
    def body(src_h, rel_h, dst_h, cg_h, cd_h, cnt_h,
             src_s0, src_s1, rel_s0, rel_s1, dst_s0, dst_s1,
             ccg, ccd, cbuf, sem0, sem1):
        src_s = (src_s0, src_s1)
        rel_s = (rel_s0, rel_s1)
        dst_s = (dst_s0, dst_s1)
        cid = lax.axis_index("c")
        sid = lax.axis_index("s")
        sems = (sem0, sem1)
        e0 = sid * EPS
        lo = cid * HALF

        def fire(m, b):
            base = pl.multiple_of(e0 + m * SCH, 8)
            pltpu.async_copy(src_h.at[pl.ds(base, SCH)], src_s[b], sems[b])
            pltpu.async_copy(rel_h.at[pl.ds(base, SCH)], rel_s[b], sems[b])
            pltpu.async_copy(dst_h.at[pl.ds(base, SCH)], dst_s[b], sems[b])

        def wait(b):
            for ref in (src_s, rel_s, dst_s):
                pltpu.make_async_copy(src_h.at[pl.ds(0, SCH)], ref[b],
                                      sems[b]).wait()

        fire(0, 0)
        iota = lax.iota(jnp.int32, 16)

        def pre(i, c):
            sl = pl.ds(i * 16, 16)
            ccg[sl] = (iota + i * 16) & 8191
            ccd[sl] = HALF + ((iota + i) & 63)
            return c
        lax.fori_loop(0, CAP // 16, pre, 0)

        def mega(i, fillv):
            for b in range(2):
                m = i * 2 + b
                wait(b)

                @pl.when(m + 1 < NMEGA)
                def _():
                    fire(m + 1, 1 - b)

                def scan(v, fv):
                    sl = pl.ds(v * 16, 16)
                    s = src_s[b][sl]
                    r = rel_s[b][sl]
                    d = dst_s[b][sl] - lo
                    g = s * R + r
                    keep = (d >= 0) & (d < HALF)
                    ki = keep.astype(jnp.int32)
                    pos = fv * 16 + iota
                    plsc.store_scatter(ccg, [pos], g, mask=keep)
                    plsc.store_scatter(ccd, [pos], d, mask=keep)
                    return fv + ki
                fillv = lax.fori_loop(0, SCH // 16, scan, fillv)
            return fillv
        fillv = lax.fori_loop(0, NMEGA // 2, mega, jnp.zeros(16, jnp.int32))

        mx = fillv[0]
        for l in range(1, 16):
            mx = jnp.maximum(mx, fillv[l])
        nchunks = (mx * 16 + CH - 1) // CH

        obase = (cid * NS + sid) * CAP
        pltpu.sync_copy(ccg, cg_h.at[pl.ds(obase, CAP)])
        pltpu.sync_copy(ccd, cd_h.at[pl.ds(obase, CAP)])
        cbuf[...] = jnp.full((16,), nchunks, jnp.int32)
        pltpu.sync_copy(cbuf, cnt_h.at[pl.ds((cid * NS + sid) * 16, 16)])

    return body(src, rel, dst)


def _edge_scatter(cg, cd, cnt, z, zero):
    mesh = plsc.VectorSubcoreMesh(core_axis_name="c", subcore_axis_name="s")

    @functools.partial(
        pl.kernel,
        mesh=mesh,
        compiler_params=pltpu.CompilerParams(needs_layout_passes=False),
        out_type=jax.ShapeDtypeStruct((NC, NPH, D), jnp.float32),
        scratch_types=[
            pltpu.VMEM((NIB, CH), jnp.int32),
            pltpu.VMEM((NIB, CH), jnp.int32),
            pltpu.VMEM((NRB, CH, D), jnp.float32),
            pltpu.VMEM((16,), jnp.int32),
            pltpu.VMEM_SHARED((NPH, D), jnp.float32),
        ] + [pltpu.SemaphoreType.DMA] * (NIB + 2 * NRB + 1),
    )
    def body(cg_h, cd_h, cnt_h, z_h, zero_h, out_h,
             gbuf, dbuf, rows_v, cbuf, acc, *sems):
        sem_i = sems[:NIB]
        sem_g = sems[NIB:NIB + NRB]
        sem_s = sems[NIB + NRB:NIB + 2 * NRB]
        sem_c = sems[NIB + 2 * NRB]
        cid = lax.axis_index("c")
        sid = lax.axis_index("s")
        rpt = NPH // NS
        r0 = sid * rpt
        obase = (cid * NS + sid) * CAP

        pltpu.async_copy(cnt_h.at[pl.ds((cid * NS + sid) * 16, 16)], cbuf,
                         sem_c)
        pltpu.sync_copy(zero_h.at[pl.ds(r0, rpt)], acc.at[pl.ds(r0, rpt)])
        pltpu.make_async_copy(cnt_h.at[pl.ds(0, 16)], cbuf, sem_c).wait()
        nchunks = cbuf[pl.ds(0, 16)][0]
        plsc.subcore_barrier()

        def fire_idx(k, ib):
            base = pl.multiple_of(obase + k * CH, 8)
            pltpu.async_copy(cg_h.at[pl.ds(base, CH)], gbuf.at[ib], sem_i[ib])
            pltpu.async_copy(cd_h.at[pl.ds(base, CH)], dbuf.at[ib], sem_i[ib])

        def wait_idx(ib):
            pltpu.make_async_copy(cg_h.at[pl.ds(0, CH)], gbuf.at[ib],
                                  sem_i[ib]).wait()
            pltpu.make_async_copy(cg_h.at[pl.ds(0, CH)], dbuf.at[ib],
                                  sem_i[ib]).wait()

        def fire_gather(ib, rb):
            pltpu.async_copy(z_h.at[gbuf.at[ib]], rows_v.at[rb], sem_g[rb])

        def wait_gather(rb):
            pltpu.make_async_copy(z_h.at[pl.ds(0, CH)], rows_v.at[rb],
                                  sem_g[rb]).wait()

        def fire_scatter(rb, ib):
            pltpu.async_copy(rows_v.at[rb], acc.at[dbuf.at[ib]], sem_s[rb],
                             add=True)

        def wait_scatter(rb):
            pltpu.make_async_copy(rows_v.at[rb], acc.at[pl.ds(0, CH)],
                                  sem_s[rb]).wait()

        for u in range(3):
            @pl.when(u < nchunks)
            def _():
                fire_idx(u, u)

        @pl.when(0 < nchunks)
        def _():
            wait_idx(0)
            fire_gather(0, 0)

        def batch(i, c):
            for u in range(4):
                k = i * 4 + u
                rb = u % 2
                ib = u

                @pl.when((k >= 1) & (k - 1 < nchunks))
                def _():
                    wait_scatter((u + 1) % 2)

                @pl.when(k + 3 < nchunks)
                def _():
                    fire_idx(k + 3, (u + 3) % 4)

                @pl.when(k + 1 < nchunks)
                def _():
                    wait_idx((u + 1) % 4)
                    fire_gather((u + 1) % 4, (u + 1) % 2)

                @pl.when(k < nchunks)
                def _():
                    wait_gather(rb)
                    fire_scatter(rb, ib)
            return c
        lax.fori_loop(0, NBATCH, batch, 0)

        plsc.subcore_barrier()
        pltpu.sync_copy(acc.at[pl.ds(r0, rpt)], out_h.at[cid, pl.ds(r0, rpt)])

    return body(cg, cd, cnt, z, zero)


def kernel(input, edge_index, edge_relation, Ws0, bs0, Wl0, bl0,
           Ws1, bs1, Wl1, bl1):
    x = input
    src = edge_index[0]
    dst = edge_index[1]
    rel = edge_relation
    zero = jnp.zeros((NPH, D), jnp.float32)

    cg, cd, cnt = _edge_prep(src, rel, dst)
    z1 = _transform1(x, Wl0.T, bl0.reshape(1, D * R)).reshape(N * R, D)
    p1 = _edge_scatter(cg, cd, cnt, z1, zero)
    z2 = _transform2(p1, Wl1.T, bl1.reshape(1, D * R)).reshape(N * R, D)
    p2 = _edge_scatter(cg, cd, cnt, z2, zero)
    return _finalize(p2)


# speedup vs baseline: 14.7531x; 1.0166x over previous
import functools

import jax
import jax.numpy as jnp
from jax import lax
from jax.experimental import pallas as pl
from jax.experimental.pallas import tpu as pltpu
from jax.experimental.pallas import tpu_sc as plsc

N = 10000
E = 320000
D = 128
R = 7

NC = 2
NS = 16
HALF = N // 2
NPH = 5120
EPS = E // NS
SCH = 2000
NMEGA = EPS // SCH
CAP = 20064
CH = 88
CAPCH = CAP // CH
NIB = 4
NRB = 2
NBATCH = 58

BN = 1000


def _transform1(x, Wt, b2):
    def body(x_ref, w_ref, b_ref, o_ref):
        o_ref[...] = jnp.dot(x_ref[...], w_ref[...],
                             preferred_element_type=jnp.float32) + b_ref[...]
    return pl.pallas_call(
        body,
        grid=(N // BN,),
        in_specs=[
            pl.BlockSpec((BN, D), lambda i: (i, 0)),
            pl.BlockSpec((D, D * R), lambda i: (0, 0)),
            pl.BlockSpec((1, D * R), lambda i: (0, 0)),
        ],
        out_specs=pl.BlockSpec((BN, D * R), lambda i: (i, 0)),
        out_shape=jax.ShapeDtypeStruct((N, D * R), jnp.float32),
    )(x, Wt, b2)


def _transform2(p, Wt, b2):
    nb = N // BN // NC

    def body(p_ref, w_ref, b_ref, o_ref):
        x = jnp.maximum(p_ref[0], 0.0)
        o_ref[...] = jnp.dot(x, w_ref[...],
                             preferred_element_type=jnp.float32) + b_ref[...]
    return pl.pallas_call(
        body,
        grid=(N // BN,),
        in_specs=[
            pl.BlockSpec((1, BN, D), lambda i: (i // nb, i % nb, 0)),
            pl.BlockSpec((D, D * R), lambda i: (0, 0)),
            pl.BlockSpec((1, D * R), lambda i: (0, 0)),
        ],
        out_specs=pl.BlockSpec((BN, D * R), lambda i: (i, 0)),
        out_shape=jax.ShapeDtypeStruct((N, D * R), jnp.float32),
    )(p, Wt, b2)


def _finalize(p):
    nb = N // BN // NC

    def body(p_ref, g_ref, n_ref):
        i = pl.program_id(0)
        x = jnp.maximum(p_ref[0], 0.0)
        n_ref[...] = x

        @pl.when(i == 0)
        def _():
            g_ref[...] = jnp.zeros_like(g_ref)

        g_ref[...] += jnp.sum(x, axis=0, keepdims=True)

    graph, node = pl.pallas_call(
        body,
        grid=(N // BN,),
        in_specs=[pl.BlockSpec((1, BN, D), lambda i: (i // nb, i % nb, 0))],
        out_specs=[pl.BlockSpec((1, D), lambda i: (0, 0)),
                   pl.BlockSpec((BN, D), lambda i: (i, 0))],
        out_shape=[jax.ShapeDtypeStruct((1, D), jnp.float32),
                   jax.ShapeDtypeStruct((N, D), jnp.float32)],
    )(p)
    return graph.reshape(D), node


def _edge_prep(src, rel, dst):
    mesh = plsc.VectorSubcoreMesh(core_axis_name="c", subcore_axis_name="s")

    @functools.partial(
        pl.kernel,
        mesh=mesh,
        compiler_params=pltpu.CompilerParams(needs_layout_passes=False),
        out_type=(
            jax.ShapeDtypeStruct((NC * NS * CAP,), jnp.int32),
            jax.ShapeDtypeStruct((NC * NS * CAP,), jnp.int32),
            jax.ShapeDtypeStruct((NC * NS * 16,), jnp.int32),
        ),
        scratch_types=[
            pltpu.VMEM((SCH,), jnp.int32),
            pltpu.VMEM((SCH,), jnp.int32),
            pltpu.VMEM((SCH,), jnp.int32),
            pltpu.VMEM((SCH,), jnp.int32),
            pltpu.VMEM((SCH,), jnp.int32),
            pltpu.VMEM((SCH,), jnp.int32),
            pltpu.VMEM((CAP,), jnp.int32),
            pltpu.VMEM((CAP,), jnp.int32),
            pltpu.VMEM((16,), jnp.int32),
            pltpu.SemaphoreType.DMA,
            pltpu.SemaphoreType.DMA,
        ],
    )
    def body(src_h, rel_h, dst_h, cg_h, cd_h, cnt_h,
             src_s0, src_s1, rel_s0, rel_s1, dst_s0, dst_s1,
             ccg, ccd, cbuf, sem0, sem1):
        src_s = (src_s0, src_s1)
        rel_s = (rel_s0, rel_s1)
        dst_s = (dst_s0, dst_s1)
        cid = lax.axis_index("c")
        sid = lax.axis_index("s")
        sems = (sem0, sem1)
        e0 = sid * EPS
        lo = cid * HALF

        def fire(m, b):
            base = pl.multiple_of(e0 + m * SCH, 8)
            pltpu.async_copy(src_h.at[pl.ds(base, SCH)], src_s[b], sems[b])
            pltpu.async_copy(rel_h.at[pl.ds(base, SCH)], rel_s[b], sems[b])
            pltpu.async_copy(dst_h.at[pl.ds(base, SCH)], dst_s[b], sems[b])

        def wait(b):
            for ref in (src_s, rel_s, dst_s):
                pltpu.make_async_copy(src_h.at[pl.ds(0, SCH)], ref[b],
                                      sems[b]).wait()

        fire(0, 0)
        iota = lax.iota(jnp.int32, 16)

        def pre(i, c):
            sl = pl.ds(i * 16, 16)
            ccg[sl] = (iota + i * 16) & 8191
            ccd[sl] = HALF + ((iota + i) & 63)
            return c
        lax.fori_loop(0, CAP // 16, pre, 0)

        def mega(i, fillv):
            for b in range(2):
                m = i * 2 + b
                wait(b)

                @pl.when(m + 1 < NMEGA)
                def _():
                    fire(m + 1, 1 - b)

                def scan(v, fv):
                    sl = pl.ds(v * 16, 16)
                    s = src_s[b][sl]
                    r = rel_s[b][sl]
                    d = dst_s[b][sl] - lo
                    g = s * R + r
                    keep = (d >= 0) & (d < HALF)
                    ki = keep.astype(jnp.int32)
                    pos = fv * 16 + iota
                    plsc.store_scatter(ccg, [pos], g, mask=keep)
                    plsc.store_scatter(ccd, [pos], d, mask=keep)
                    return fv + ki
                fillv = lax.fori_loop(0, SCH // 16, scan, fillv)
            return fillv
        fillv = lax.fori_loop(0, NMEGA // 2, mega, jnp.zeros(16, jnp.int32))

        mx = fillv[0]
        for l in range(1, 16):
            mx = jnp.maximum(mx, fillv[l])
        nchunks = (mx * 16 + CH - 1) // CH

        obase = (cid * NS + sid) * CAP
        pltpu.sync_copy(ccg, cg_h.at[pl.ds(obase, CAP)])
        pltpu.sync_copy(ccd, cd_h.at[pl.ds(obase, CAP)])
        cbuf[...] = jnp.full((16,), nchunks, jnp.int32)
        pltpu.sync_copy(cbuf, cnt_h.at[pl.ds((cid * NS + sid) * 16, 16)])

    return body(src, rel, dst)


def _edge_scatter(cg, cd, cnt, z, zero, final=False):
    mesh = plsc.VectorSubcoreMesh(core_axis_name="c", subcore_axis_name="s")

    if final:
        out_type = (jax.ShapeDtypeStruct((N, D), jnp.float32),
                    jax.ShapeDtypeStruct((NC * NS * D,), jnp.float32))
    else:
        out_type = jax.ShapeDtypeStruct((NC, NPH, D), jnp.float32)

    @functools.partial(
        pl.kernel,
        mesh=mesh,
        compiler_params=pltpu.CompilerParams(needs_layout_passes=False),
        out_type=out_type,
        scratch_types=[
            pltpu.VMEM((NIB, CH), jnp.int32),
            pltpu.VMEM((NIB, CH), jnp.int32),
            pltpu.VMEM((NRB, CH, D), jnp.float32),
            pltpu.VMEM((16,), jnp.int32),
            pltpu.VMEM((D,), jnp.float32),
            pltpu.VMEM_SHARED((NPH, D), jnp.float32),
        ] + [pltpu.SemaphoreType.DMA] * (NIB + 2 * NRB + 1),
    )
    def body(cg_h, cd_h, cnt_h, z_h, zero_h, *rest):
        if final:
            node_h, gp_h = rest[0], rest[1]
            gbuf, dbuf, rows_v, cbuf, cbuf2, acc = rest[2:8]
            sems = rest[8:]
        else:
            out_h = rest[0]
            gbuf, dbuf, rows_v, cbuf, cbuf2, acc = rest[1:7]
            sems = rest[7:]
        sem_i = sems[:NIB]
        sem_g = sems[NIB:NIB + NRB]
        sem_s = sems[NIB + NRB:NIB + 2 * NRB]
        sem_c = sems[NIB + 2 * NRB]
        cid = lax.axis_index("c")
        sid = lax.axis_index("s")
        rpt = NPH // NS
        r0 = sid * rpt
        obase = (cid * NS + sid) * CAP

        pltpu.async_copy(cnt_h.at[pl.ds((cid * NS + sid) * 16, 16)], cbuf,
                         sem_c)
        pltpu.sync_copy(zero_h.at[pl.ds(r0, rpt)], acc.at[pl.ds(r0, rpt)])
        pltpu.make_async_copy(cnt_h.at[pl.ds(0, 16)], cbuf, sem_c).wait()
        nchunks = cbuf[pl.ds(0, 16)][0]
        plsc.subcore_barrier()

        def fire_idx(k, ib):
            base = pl.multiple_of(obase + k * CH, 8)
            pltpu.async_copy(cg_h.at[pl.ds(base, CH)], gbuf.at[ib], sem_i[ib])
            pltpu.async_copy(cd_h.at[pl.ds(base, CH)], dbuf.at[ib], sem_i[ib])

        def wait_idx(ib):
            pltpu.make_async_copy(cg_h.at[pl.ds(0, CH)], gbuf.at[ib],
                                  sem_i[ib]).wait()
            pltpu.make_async_copy(cg_h.at[pl.ds(0, CH)], dbuf.at[ib],
                                  sem_i[ib]).wait()

        def fire_gather(ib, rb):
            pltpu.async_copy(z_h.at[gbuf.at[ib]], rows_v.at[rb], sem_g[rb])

        def wait_gather(rb):
            pltpu.make_async_copy(z_h.at[pl.ds(0, CH)], rows_v.at[rb],
                                  sem_g[rb]).wait()

        def fire_scatter(rb, ib):
            pltpu.async_copy(rows_v.at[rb], acc.at[dbuf.at[ib]], sem_s[rb],
                             add=True)

        def wait_scatter(rb):
            pltpu.make_async_copy(rows_v.at[rb], acc.at[pl.ds(0, CH)],
                                  sem_s[rb]).wait()

        for u in range(3):
            @pl.when(u < nchunks)
            def _():
                fire_idx(u, u)

        @pl.when(0 < nchunks)
        def _():
            wait_idx(0)
            fire_gather(0, 0)

        def batch(i, c):
            for u in range(4):
                k = i * 4 + u
                rb = u % 2
                ib = u

                @pl.when((k >= 1) & (k - 1 < nchunks))
                def _():
                    wait_scatter((u + 1) % 2)

                @pl.when(k + 3 < nchunks)
                def _():
                    fire_idx(k + 3, (u + 3) % 4)

                @pl.when(k + 1 < nchunks)
                def _():
                    wait_idx((u + 1) % 4)
                    fire_gather((u + 1) % 4, (u + 1) % 2)

                @pl.when(k < nchunks)
                def _():
                    wait_gather(rb)
                    fire_scatter(rb, ib)
            return c
        lax.fori_loop(0, NBATCH, batch, 0)

        plsc.subcore_barrier()
        if not final:
            pltpu.sync_copy(acc.at[pl.ds(r0, rpt)],
                            out_h.at[cid, pl.ds(r0, rpt)])
            return

        FR = 40
        gz = [jnp.zeros(16, jnp.float32) for _ in range(D // 16)]

        def fchunk(j, gs):
            lbase = r0 + j * FR

            def work():
                pltpu.sync_copy(acc.at[pl.ds(lbase, FR)],
                                rows_v.at[0, pl.ds(0, FR)])
                gs2 = list(gs)

                def rrow(t, carry):
                    carry2 = list(carry)
                    for q in range(D // 16):
                        sl = pl.ds(q * 16, 16)
                        v = jnp.maximum(rows_v[0, t, sl], 0.0)
                        rows_v[0, t, sl] = v
                        carry2[q] = carry2[q] + v
                    return tuple(carry2)
                gs2 = lax.fori_loop(0, FR, rrow, tuple(gs2))
                pltpu.sync_copy(
                    rows_v.at[0, pl.ds(0, FR)],
                    node_h.at[pl.ds(cid * HALF + lbase, FR)])
                return gs2
            return lax.cond(lbase < HALF, work, lambda: gs)
        gz = lax.fori_loop(0, rpt // FR, fchunk, tuple(gz))

        for q in range(D // 16):
            cbuf2[pl.ds(q * 16, 16)] = gz[q]
        pltpu.sync_copy(cbuf2,
                        gp_h.at[pl.ds((cid * NS + sid) * D, D)])

    return body(cg, cd, cnt, z, zero)


def kernel(input, edge_index, edge_relation, Ws0, bs0, Wl0, bl0,
           Ws1, bs1, Wl1, bl1):
    x = input
    src = edge_index[0]
    dst = edge_index[1]
    rel = edge_relation
    zero = jnp.zeros((NPH, D), jnp.float32)

    cg, cd, cnt = _edge_prep(src, rel, dst)
    z1 = _transform1(x, Wl0.T, bl0.reshape(1, D * R)).reshape(N * R, D)
    p1 = _edge_scatter(cg, cd, cnt, z1, zero)
    z2 = _transform2(p1, Wl1.T, bl1.reshape(1, D * R)).reshape(N * R, D)
    node, gpart = _edge_scatter(cg, cd, cnt, z2, zero, final=True)
    graph = jnp.sum(gpart.reshape(NC * NS, D), axis=0)
    return graph, node


# speedup vs baseline: 15.3790x; 1.0424x over previous
import functools

import jax
import jax.numpy as jnp
from jax import lax
from jax.experimental import pallas as pl
from jax.experimental.pallas import tpu as pltpu
from jax.experimental.pallas import tpu_sc as plsc

N = 10000
E = 320000
D = 128
R = 7

NC = 2
NS = 16
HALF = N // 2
NPH = 5120
EPS = E // NS
SCH = 2000
NMEGA = EPS // SCH
CAP = 20064
CH = 88
CAPCH = CAP // CH
NIB = 4
NRB = 2
NBATCH = 58

BN = 1000


def _transform1(x, Wt, b2):
    nb = N // BN

    def body(x_ref, w_ref, b_ref, o_ref):
        r = pl.program_id(1)
        w = w_ref[:, pl.ds(r * D, D)]
        b = b_ref[0, pl.ds(r * D, D)]
        o_ref[...] = jnp.dot(x_ref[...], w,
                             preferred_element_type=jnp.float32) + b
    return pl.pallas_call(
        body,
        grid=(nb, R),
        in_specs=[
            pl.BlockSpec((BN, D), lambda i, r: (i, 0)),
            pl.BlockSpec((D, D * R), lambda i, r: (0, 0)),
            pl.BlockSpec((1, D * R), lambda i, r: (0, 0)),
        ],
        out_specs=pl.BlockSpec((BN, D), lambda i, r: (r * nb + i, 0)),
        out_shape=jax.ShapeDtypeStruct((R * N, D), jnp.float32),
    )(x, Wt, b2)


def _transform2(p, Wt, b2):
    nb = N // BN
    nbh = nb // NC

    def body(p_ref, w_ref, b_ref, o_ref):
        r = pl.program_id(1)
        w = w_ref[:, pl.ds(r * D, D)]
        b = b_ref[0, pl.ds(r * D, D)]
        x = jnp.maximum(p_ref[0], 0.0)
        o_ref[...] = jnp.dot(x, w,
                             preferred_element_type=jnp.float32) + b
    return pl.pallas_call(
        body,
        grid=(nb, R),
        in_specs=[
            pl.BlockSpec((1, BN, D), lambda i, r: (i // nbh, i % nbh, 0)),
            pl.BlockSpec((D, D * R), lambda i, r: (0, 0)),
            pl.BlockSpec((1, D * R), lambda i, r: (0, 0)),
        ],
        out_specs=pl.BlockSpec((BN, D), lambda i, r: (r * nb + i, 0)),
        out_shape=jax.ShapeDtypeStruct((R * N, D), jnp.float32),
    )(p, Wt, b2)


def _finalize(p):
    nb = N // BN // NC

    def body(p_ref, g_ref, n_ref):
        i = pl.program_id(0)
        x = jnp.maximum(p_ref[0], 0.0)
        n_ref[...] = x

        @pl.when(i == 0)
        def _():
            g_ref[...] = jnp.zeros_like(g_ref)

        g_ref[...] += jnp.sum(x, axis=0, keepdims=True)

    graph, node = pl.pallas_call(
        body,
        grid=(N // BN,),
        in_specs=[pl.BlockSpec((1, BN, D), lambda i: (i // nb, i % nb, 0))],
        out_specs=[pl.BlockSpec((1, D), lambda i: (0, 0)),
                   pl.BlockSpec((BN, D), lambda i: (i, 0))],
        out_shape=[jax.ShapeDtypeStruct((1, D), jnp.float32),
                   jax.ShapeDtypeStruct((N, D), jnp.float32)],
    )(p)
    return graph.reshape(D), node


def _edge_prep(src, rel, dst):
    mesh = plsc.VectorSubcoreMesh(core_axis_name="c", subcore_axis_name="s")

    @functools.partial(
        pl.kernel,
        mesh=mesh,
        compiler_params=pltpu.CompilerParams(needs_layout_passes=False),
        out_type=(
            jax.ShapeDtypeStruct((NC * NS * CAP,), jnp.int32),
            jax.ShapeDtypeStruct((NC * NS * CAP,), jnp.int32),
            jax.ShapeDtypeStruct((NC * NS * 16,), jnp.int32),
        ),
        scratch_types=[
            pltpu.VMEM((SCH,), jnp.int32),
            pltpu.VMEM((SCH,), jnp.int32),
            pltpu.VMEM((SCH,), jnp.int32),
            pltpu.VMEM((SCH,), jnp.int32),
            pltpu.VMEM((SCH,), jnp.int32),
            pltpu.VMEM((SCH,), jnp.int32),
            pltpu.VMEM((CAP,), jnp.int32),
            pltpu.VMEM((CAP,), jnp.int32),
            pltpu.VMEM((16,), jnp.int32),
            pltpu.SemaphoreType.DMA,
            pltpu.SemaphoreType.DMA,
        ],
    )
    def body(src_h, rel_h, dst_h, cg_h, cd_h, cnt_h,
             src_s0, src_s1, rel_s0, rel_s1, dst_s0, dst_s1,
             ccg, ccd, cbuf, sem0, sem1):
        src_s = (src_s0, src_s1)
        rel_s = (rel_s0, rel_s1)
        dst_s = (dst_s0, dst_s1)
        cid = lax.axis_index("c")
        sid = lax.axis_index("s")
        sems = (sem0, sem1)
        e0 = sid * EPS
        lo = cid * HALF

        def fire(m, b):
            base = pl.multiple_of(e0 + m * SCH, 8)
            pltpu.async_copy(src_h.at[pl.ds(base, SCH)], src_s[b], sems[b])
            pltpu.async_copy(rel_h.at[pl.ds(base, SCH)], rel_s[b], sems[b])
            pltpu.async_copy(dst_h.at[pl.ds(base, SCH)], dst_s[b], sems[b])

        def wait(b):
            for ref in (src_s, rel_s, dst_s):
                pltpu.make_async_copy(src_h.at[pl.ds(0, SCH)], ref[b],
                                      sems[b]).wait()

        fire(0, 0)
        iota = lax.iota(jnp.int32, 16)

        def pre(i, c):
            sl = pl.ds(i * 16, 16)
            ccg[sl] = (iota + i * 16) & 8191
            ccd[sl] = HALF + ((iota + i) & 63)
            return c
        lax.fori_loop(0, CAP // 16, pre, 0)

        def mega(i, fillv):
            for b in range(2):
                m = i * 2 + b
                wait(b)

                @pl.when(m + 1 < NMEGA)
                def _():
                    fire(m + 1, 1 - b)

                def scan(v, fv):
                    sl = pl.ds(v * 16, 16)
                    s = src_s[b][sl]
                    r = rel_s[b][sl]
                    d = dst_s[b][sl] - lo
                    g = r * N + s
                    keep = (d >= 0) & (d < HALF)
                    ki = keep.astype(jnp.int32)
                    pos = fv * 16 + iota
                    plsc.store_scatter(ccg, [pos], g, mask=keep)
                    plsc.store_scatter(ccd, [pos], d, mask=keep)
                    return fv + ki
                fillv = lax.fori_loop(0, SCH // 16, scan, fillv)
            return fillv
        fillv = lax.fori_loop(0, NMEGA // 2, mega, jnp.zeros(16, jnp.int32))

        mx = fillv[0]
        for l in range(1, 16):
            mx = jnp.maximum(mx, fillv[l])
        nchunks = (mx * 16 + CH - 1) // CH

        obase = (cid * NS + sid) * CAP
        pltpu.sync_copy(ccg, cg_h.at[pl.ds(obase, CAP)])
        pltpu.sync_copy(ccd, cd_h.at[pl.ds(obase, CAP)])
        cbuf[...] = jnp.full((16,), nchunks, jnp.int32)
        pltpu.sync_copy(cbuf, cnt_h.at[pl.ds((cid * NS + sid) * 16, 16)])

    return body(src, rel, dst)


def _edge_scatter(cg, cd, cnt, z, zero, final=False):
    mesh = plsc.VectorSubcoreMesh(core_axis_name="c", subcore_axis_name="s")

    if final:
        out_type = (jax.ShapeDtypeStruct((N, D), jnp.float32),
                    jax.ShapeDtypeStruct((NC * NS * D,), jnp.float32))
    else:
        out_type = jax.ShapeDtypeStruct((NC, NPH, D), jnp.float32)

    @functools.partial(
        pl.kernel,
        mesh=mesh,
        compiler_params=pltpu.CompilerParams(needs_layout_passes=False),
        out_type=out_type,
        scratch_types=[
            pltpu.VMEM((NIB, CH), jnp.int32),
            pltpu.VMEM((NIB, CH), jnp.int32),
            pltpu.VMEM((NRB, CH, D), jnp.float32),
            pltpu.VMEM((16,), jnp.int32),
            pltpu.VMEM((D,), jnp.float32),
            pltpu.VMEM_SHARED((NPH, D), jnp.float32),
        ] + [pltpu.SemaphoreType.DMA] * (NIB + 2 * NRB + 1),
    )
    def body(cg_h, cd_h, cnt_h, z_h, zero_h, *rest):
        if final:
            node_h, gp_h = rest[0], rest[1]
            gbuf, dbuf, rows_v, cbuf, cbuf2, acc = rest[2:8]
            sems = rest[8:]
        else:
            out_h = rest[0]
            gbuf, dbuf, rows_v, cbuf, cbuf2, acc = rest[1:7]
            sems = rest[7:]
        sem_i = sems[:NIB]
        sem_g = sems[NIB:NIB + NRB]
        sem_s = sems[NIB + NRB:NIB + 2 * NRB]
        sem_c = sems[NIB + 2 * NRB]
        cid = lax.axis_index("c")
        sid = lax.axis_index("s")
        rpt = NPH // NS
        r0 = sid * rpt
        obase = (cid * NS + sid) * CAP

        pltpu.async_copy(cnt_h.at[pl.ds((cid * NS + sid) * 16, 16)], cbuf,
                         sem_c)
        pltpu.sync_copy(zero_h.at[pl.ds(r0, rpt)], acc.at[pl.ds(r0, rpt)])
        pltpu.make_async_copy(cnt_h.at[pl.ds(0, 16)], cbuf, sem_c).wait()
        nchunks = cbuf[pl.ds(0, 16)][0]
        plsc.subcore_barrier()

        def fire_idx(k, ib):
            base = pl.multiple_of(obase + k * CH, 8)
            pltpu.async_copy(cg_h.at[pl.ds(base, CH)], gbuf.at[ib], sem_i[ib])
            pltpu.async_copy(cd_h.at[pl.ds(base, CH)], dbuf.at[ib], sem_i[ib])

        def wait_idx(ib):
            pltpu.make_async_copy(cg_h.at[pl.ds(0, CH)], gbuf.at[ib],
                                  sem_i[ib]).wait()
            pltpu.make_async_copy(cg_h.at[pl.ds(0, CH)], dbuf.at[ib],
                                  sem_i[ib]).wait()

        def fire_gather(ib, rb):
            pltpu.async_copy(z_h.at[gbuf.at[ib]], rows_v.at[rb], sem_g[rb])

        def wait_gather(rb):
            pltpu.make_async_copy(z_h.at[pl.ds(0, CH)], rows_v.at[rb],
                                  sem_g[rb]).wait()

        def fire_scatter(rb, ib):
            pltpu.async_copy(rows_v.at[rb], acc.at[dbuf.at[ib]], sem_s[rb],
                             add=True)

        def wait_scatter(rb):
            pltpu.make_async_copy(rows_v.at[rb], acc.at[pl.ds(0, CH)],
                                  sem_s[rb]).wait()

        for u in range(3):
            @pl.when(u < nchunks)
            def _():
                fire_idx(u, u)

        @pl.when(0 < nchunks)
        def _():
            wait_idx(0)
            fire_gather(0, 0)

        def batch(i, c):
            for u in range(4):
                k = i * 4 + u
                rb = u % 2
                ib = u

                @pl.when((k >= 1) & (k - 1 < nchunks))
                def _():
                    wait_scatter((u + 1) % 2)

                @pl.when(k + 3 < nchunks)
                def _():
                    fire_idx(k + 3, (u + 3) % 4)

                @pl.when(k + 1 < nchunks)
                def _():
                    wait_idx((u + 1) % 4)
                    fire_gather((u + 1) % 4, (u + 1) % 2)

                @pl.when(k < nchunks)
                def _():
                    wait_gather(rb)
                    fire_scatter(rb, ib)
            return c
        lax.fori_loop(0, NBATCH, batch, 0)

        plsc.subcore_barrier()
        if not final:
            pltpu.sync_copy(acc.at[pl.ds(r0, rpt)],
                            out_h.at[cid, pl.ds(r0, rpt)])
            return

        FR = 40
        gz = [jnp.zeros(16, jnp.float32) for _ in range(D // 16)]

        def fchunk(j, gs):
            lbase = r0 + j * FR

            def work():
                pltpu.sync_copy(acc.at[pl.ds(lbase, FR)],
                                rows_v.at[0, pl.ds(0, FR)])
                gs2 = list(gs)

                def rrow(t, carry):
                    carry2 = list(carry)
                    for q in range(D // 16):
                        sl = pl.ds(q * 16, 16)
                        v = jnp.maximum(rows_v[0, t, sl], 0.0)
                        rows_v[0, t, sl] = v
                        carry2[q] = carry2[q] + v
                    return tuple(carry2)
                gs2 = lax.fori_loop(0, FR, rrow, tuple(gs2))
                pltpu.sync_copy(
                    rows_v.at[0, pl.ds(0, FR)],
                    node_h.at[pl.ds(cid * HALF + lbase, FR)])
                return gs2
            return lax.cond(lbase < HALF, work, lambda: gs)
        gz = lax.fori_loop(0, rpt // FR, fchunk, tuple(gz))

        for q in range(D // 16):
            cbuf2[pl.ds(q * 16, 16)] = gz[q]
        pltpu.sync_copy(cbuf2,
                        gp_h.at[pl.ds((cid * NS + sid) * D, D)])

    return body(cg, cd, cnt, z, zero)


def kernel(input, edge_index, edge_relation, Ws0, bs0, Wl0, bl0,
           Ws1, bs1, Wl1, bl1):
    x = input
    src = edge_index[0]
    dst = edge_index[1]
    rel = edge_relation
    zero = jnp.zeros((NPH, D), jnp.float32)

    cg, cd, cnt = _edge_prep(src, rel, dst)
    z1 = _transform1(x, Wl0.T, bl0.reshape(1, D * R))
    p1 = _edge_scatter(cg, cd, cnt, z1, zero)
    z2 = _transform2(p1, Wl1.T, bl1.reshape(1, D * R))
    node, gpart = _edge_scatter(cg, cd, cnt, z2, zero, final=True)
    graph = jnp.sum(gpart.reshape(NC * NS, D), axis=0)
    return graph, node


# speedup vs baseline: 17.8015x; 1.1575x over previous
import functools

import jax
import jax.numpy as jnp
from jax import lax
from jax.experimental import pallas as pl
from jax.experimental.pallas import tpu as pltpu
from jax.experimental.pallas import tpu_sc as plsc

N = 10000
E = 320000
D = 128
R = 7

NC = 2
NS = 16
HALF = N // 2
NPH = 5120
EPS = E // NS
SCH = 2000
NMEGA = EPS // SCH
CAP = 20064
CH = 88
CAPCH = CAP // CH
NIB = 4
NRB = 2
NBATCH = 58

BN = 5000


def _transform1(x, Wl, b2):
    nb = N // BN

    def body(x_ref, w_ref, b_ref, o_ref):
        r = pl.program_id(1)
        w = w_ref[pl.ds(r * D, D), :]
        b = b_ref[0, pl.ds(r * D, D)]
        o_ref[...] = lax.dot_general(
            x_ref[...], w, (((1,), (1,)), ((), ())),
            preferred_element_type=jnp.float32) + b
    return pl.pallas_call(
        body,
        grid=(nb, R),
        in_specs=[
            pl.BlockSpec((BN, D), lambda i, r: (i, 0)),
            pl.BlockSpec((D * R, D), lambda i, r: (0, 0)),
            pl.BlockSpec((1, D * R), lambda i, r: (0, 0)),
        ],
        out_specs=pl.BlockSpec((BN, D), lambda i, r: (r * nb + i, 0)),
        out_shape=jax.ShapeDtypeStruct((R * N, D), jnp.float32),
    )(x, Wl, b2)


def _transform2(p, Wl, b2):
    nb = N // BN
    nbh = nb // NC

    def body(p_ref, w_ref, b_ref, o_ref):
        r = pl.program_id(1)
        w = w_ref[pl.ds(r * D, D), :]
        b = b_ref[0, pl.ds(r * D, D)]
        x = jnp.maximum(p_ref[0], 0.0)
        o_ref[...] = lax.dot_general(
            x, w, (((1,), (1,)), ((), ())),
            preferred_element_type=jnp.float32) + b
    return pl.pallas_call(
        body,
        grid=(nb, R),
        in_specs=[
            pl.BlockSpec((1, BN, D), lambda i, r: (i // nbh, i % nbh, 0)),
            pl.BlockSpec((D * R, D), lambda i, r: (0, 0)),
            pl.BlockSpec((1, D * R), lambda i, r: (0, 0)),
        ],
        out_specs=pl.BlockSpec((BN, D), lambda i, r: (r * nb + i, 0)),
        out_shape=jax.ShapeDtypeStruct((R * N, D), jnp.float32),
    )(p, Wl, b2)


def _finalize(p):
    nb = N // BN // NC

    def body(p_ref, g_ref, n_ref):
        i = pl.program_id(0)
        x = jnp.maximum(p_ref[0], 0.0)
        n_ref[...] = x

        @pl.when(i == 0)
        def _():
            g_ref[...] = jnp.zeros_like(g_ref)

        g_ref[...] += jnp.sum(x, axis=0, keepdims=True)

    graph, node = pl.pallas_call(
        body,
        grid=(N // BN,),
        in_specs=[pl.BlockSpec((1, BN, D), lambda i: (i // nb, i % nb, 0))],
        out_specs=[pl.BlockSpec((1, D), lambda i: (0, 0)),
                   pl.BlockSpec((BN, D), lambda i: (i, 0))],
        out_shape=[jax.ShapeDtypeStruct((1, D), jnp.float32),
                   jax.ShapeDtypeStruct((N, D), jnp.float32)],
    )(p)
    return graph.reshape(D), node


def _edge_prep(src, rel, dst):
    mesh = plsc.VectorSubcoreMesh(core_axis_name="c", subcore_axis_name="s")

    @functools.partial(
        pl.kernel,
        mesh=mesh,
        compiler_params=pltpu.CompilerParams(needs_layout_passes=False),
        out_type=(
            jax.ShapeDtypeStruct((NC * NS * CAP,), jnp.int32),
            jax.ShapeDtypeStruct((NC * NS * CAP,), jnp.int32),
            jax.ShapeDtypeStruct((NC * NS * 16,), jnp.int32),
        ),
        scratch_types=[
            pltpu.VMEM((SCH,), jnp.int32),
            pltpu.VMEM((SCH,), jnp.int32),
            pltpu.VMEM((SCH,), jnp.int32),
            pltpu.VMEM((SCH,), jnp.int32),
            pltpu.VMEM((SCH,), jnp.int32),
            pltpu.VMEM((SCH,), jnp.int32),
            pltpu.VMEM((CAP,), jnp.int32),
            pltpu.VMEM((CAP,), jnp.int32),
            pltpu.VMEM((16,), jnp.int32),
            pltpu.SemaphoreType.DMA,
            pltpu.SemaphoreType.DMA,
        ],
    )
    def body(src_h, rel_h, dst_h, cg_h, cd_h, cnt_h,
             src_s0, src_s1, rel_s0, rel_s1, dst_s0, dst_s1,
             ccg, ccd, cbuf, sem0, sem1):
        src_s = (src_s0, src_s1)
        rel_s = (rel_s0, rel_s1)
        dst_s = (dst_s0, dst_s1)
        cid = lax.axis_index("c")
        sid = lax.axis_index("s")
        sems = (sem0, sem1)
        e0 = sid * EPS
        lo = cid * HALF

        def fire(m, b):
            base = pl.multiple_of(e0 + m * SCH, 8)
            pltpu.async_copy(src_h.at[pl.ds(base, SCH)], src_s[b], sems[b])
            pltpu.async_copy(rel_h.at[pl.ds(base, SCH)], rel_s[b], sems[b])
            pltpu.async_copy(dst_h.at[pl.ds(base, SCH)], dst_s[b], sems[b])

        def wait(b):
            for ref in (src_s, rel_s, dst_s):
                pltpu.make_async_copy(src_h.at[pl.ds(0, SCH)], ref[b],
                                      sems[b]).wait()

        fire(0, 0)
        iota = lax.iota(jnp.int32, 16)

        def pre(i, c):
            sl = pl.ds(i * 16, 16)
            ccg[sl] = (iota + i * 16) & 8191
            ccd[sl] = HALF + ((iota + i) & 63)
            return c
        lax.fori_loop(0, CAP // 16, pre, 0)

        def mega(i, fillv):
            for b in range(2):
                m = i * 2 + b
                wait(b)

                @pl.when(m + 1 < NMEGA)
                def _():
                    fire(m + 1, 1 - b)

                def scan(v, fv):
                    sl = pl.ds(v * 16, 16)
                    s = src_s[b][sl]
                    r = rel_s[b][sl]
                    d = dst_s[b][sl] - lo
                    g = r * N + s
                    keep = (d >= 0) & (d < HALF)
                    ki = keep.astype(jnp.int32)
                    pos = fv * 16 + iota
                    plsc.store_scatter(ccg, [pos], g, mask=keep)
                    plsc.store_scatter(ccd, [pos], d, mask=keep)
                    return fv + ki
                fillv = lax.fori_loop(0, SCH // 16, scan, fillv)
            return fillv
        fillv = lax.fori_loop(0, NMEGA // 2, mega, jnp.zeros(16, jnp.int32))

        mx = fillv[0]
        for l in range(1, 16):
            mx = jnp.maximum(mx, fillv[l])
        nchunks = (mx * 16 + CH - 1) // CH

        obase = (cid * NS + sid) * CAP
        pltpu.sync_copy(ccg, cg_h.at[pl.ds(obase, CAP)])
        pltpu.sync_copy(ccd, cd_h.at[pl.ds(obase, CAP)])
        cbuf[...] = jnp.full((16,), nchunks, jnp.int32)
        pltpu.sync_copy(cbuf, cnt_h.at[pl.ds((cid * NS + sid) * 16, 16)])

    return body(src, rel, dst)


def _edge_scatter(cg, cd, cnt, z, zero, final=False):
    mesh = plsc.VectorSubcoreMesh(core_axis_name="c", subcore_axis_name="s")

    if final:
        out_type = (jax.ShapeDtypeStruct((N, D), jnp.float32),
                    jax.ShapeDtypeStruct((NC * NS * D,), jnp.float32))
    else:
        out_type = jax.ShapeDtypeStruct((NC, NPH, D), jnp.float32)

    @functools.partial(
        pl.kernel,
        mesh=mesh,
        compiler_params=pltpu.CompilerParams(needs_layout_passes=False),
        out_type=out_type,
        scratch_types=[
            pltpu.VMEM((NIB, CH), jnp.int32),
            pltpu.VMEM((NIB, CH), jnp.int32),
            pltpu.VMEM((NRB, CH, D), jnp.float32),
            pltpu.VMEM((16,), jnp.int32),
            pltpu.VMEM((D,), jnp.float32),
            pltpu.VMEM_SHARED((NPH, D), jnp.float32),
        ] + [pltpu.SemaphoreType.DMA] * (NIB + 2 * NRB + 1),
    )
    def body(cg_h, cd_h, cnt_h, z_h, zero_h, *rest):
        if final:
            node_h, gp_h = rest[0], rest[1]
            gbuf, dbuf, rows_v, cbuf, cbuf2, acc = rest[2:8]
            sems = rest[8:]
        else:
            out_h = rest[0]
            gbuf, dbuf, rows_v, cbuf, cbuf2, acc = rest[1:7]
            sems = rest[7:]
        sem_i = sems[:NIB]
        sem_g = sems[NIB:NIB + NRB]
        sem_s = sems[NIB + NRB:NIB + 2 * NRB]
        sem_c = sems[NIB + 2 * NRB]
        cid = lax.axis_index("c")
        sid = lax.axis_index("s")
        rpt = NPH // NS
        r0 = sid * rpt
        obase = (cid * NS + sid) * CAP

        pltpu.async_copy(cnt_h.at[pl.ds((cid * NS + sid) * 16, 16)], cbuf,
                         sem_c)
        pltpu.sync_copy(zero_h.at[pl.ds(r0, rpt)], acc.at[pl.ds(r0, rpt)])
        pltpu.make_async_copy(cnt_h.at[pl.ds(0, 16)], cbuf, sem_c).wait()
        nchunks = cbuf[pl.ds(0, 16)][0]
        plsc.subcore_barrier()

        def fire_idx(k, ib):
            base = pl.multiple_of(obase + k * CH, 8)
            pltpu.async_copy(cg_h.at[pl.ds(base, CH)], gbuf.at[ib], sem_i[ib])
            pltpu.async_copy(cd_h.at[pl.ds(base, CH)], dbuf.at[ib], sem_i[ib])

        def wait_idx(ib):
            pltpu.make_async_copy(cg_h.at[pl.ds(0, CH)], gbuf.at[ib],
                                  sem_i[ib]).wait()
            pltpu.make_async_copy(cg_h.at[pl.ds(0, CH)], dbuf.at[ib],
                                  sem_i[ib]).wait()

        def fire_gather(ib, rb):
            pltpu.async_copy(z_h.at[gbuf.at[ib]], rows_v.at[rb], sem_g[rb])

        def wait_gather(rb):
            pltpu.make_async_copy(z_h.at[pl.ds(0, CH)], rows_v.at[rb],
                                  sem_g[rb]).wait()

        def fire_scatter(rb, ib):
            pltpu.async_copy(rows_v.at[rb], acc.at[dbuf.at[ib]], sem_s[rb],
                             add=True)

        def wait_scatter(rb):
            pltpu.make_async_copy(rows_v.at[rb], acc.at[pl.ds(0, CH)],
                                  sem_s[rb]).wait()

        for u in range(3):
            @pl.when(u < nchunks)
            def _():
                fire_idx(u, u)

        @pl.when(0 < nchunks)
        def _():
            wait_idx(0)
            fire_gather(0, 0)

        def batch(i, c):
            for u in range(4):
                k = i * 4 + u
                rb = u % 2
                ib = u

                @pl.when((k >= 1) & (k - 1 < nchunks))
                def _():
                    wait_scatter((u + 1) % 2)

                @pl.when(k + 3 < nchunks)
                def _():
                    fire_idx(k + 3, (u + 3) % 4)

                @pl.when(k + 1 < nchunks)
                def _():
                    wait_idx((u + 1) % 4)
                    fire_gather((u + 1) % 4, (u + 1) % 2)

                @pl.when(k < nchunks)
                def _():
                    wait_gather(rb)
                    fire_scatter(rb, ib)
            return c
        lax.fori_loop(0, NBATCH, batch, 0)

        plsc.subcore_barrier()
        if not final:
            pltpu.sync_copy(acc.at[pl.ds(r0, rpt)],
                            out_h.at[cid, pl.ds(r0, rpt)])
            return

        FR = 40
        gz = [jnp.zeros(16, jnp.float32) for _ in range(D // 16)]

        def fchunk(j, gs):
            lbase = r0 + j * FR

            def work():
                pltpu.sync_copy(acc.at[pl.ds(lbase, FR)],
                                rows_v.at[0, pl.ds(0, FR)])
                gs2 = list(gs)

                def rrow(t, carry):
                    carry2 = list(carry)
                    for q in range(D // 16):
                        sl = pl.ds(q * 16, 16)
                        v = jnp.maximum(rows_v[0, t, sl], 0.0)
                        rows_v[0, t, sl] = v
                        carry2[q] = carry2[q] + v
                    return tuple(carry2)
                gs2 = lax.fori_loop(0, FR, rrow, tuple(gs2))
                pltpu.sync_copy(
                    rows_v.at[0, pl.ds(0, FR)],
                    node_h.at[pl.ds(cid * HALF + lbase, FR)])
                return gs2
            return lax.cond(lbase < HALF, work, lambda: gs)
        gz = lax.fori_loop(0, rpt // FR, fchunk, tuple(gz))

        for q in range(D // 16):
            cbuf2[pl.ds(q * 16, 16)] = gz[q]
        pltpu.sync_copy(cbuf2,
                        gp_h.at[pl.ds((cid * NS + sid) * D, D)])

    return body(cg, cd, cnt, z, zero)


def kernel(input, edge_index, edge_relation, Ws0, bs0, Wl0, bl0,
           Ws1, bs1, Wl1, bl1):
    x = input
    src = edge_index[0]
    dst = edge_index[1]
    rel = edge_relation
    zero = jnp.zeros((NPH, D), jnp.float32)

    cg, cd, cnt = _edge_prep(src, rel, dst)
    z1 = _transform1(x, Wl0, bl0.reshape(1, D * R))
    p1 = _edge_scatter(cg, cd, cnt, z1, zero)
    z2 = _transform2(p1, Wl1, bl1.reshape(1, D * R))
    node, gpart = _edge_scatter(cg, cd, cnt, z2, zero, final=True)
    graph = jnp.sum(gpart.reshape(NC * NS, D), axis=0)
    return graph, node


# speedup vs baseline: 17.9438x; 1.0080x over previous
import functools

import jax
import jax.numpy as jnp
from jax import lax
from jax.experimental import pallas as pl
from jax.experimental.pallas import tpu as pltpu
from jax.experimental.pallas import tpu_sc as plsc

N = 10000
E = 320000
D = 128
R = 7

NC = 2
NS = 16
HALF = N // 2
NPH = 5120
EPS = E // NS
SCH = 2000
NMEGA = EPS // SCH
CAP = 20064
CH = 88
CAPCH = CAP // CH
NIB = 4
NRB = 2
NBATCH = 58

BN = 5000


def _transform1(x, Wl, b2):
    nb = N // BN

    def body(x_ref, w_ref, b_ref, o_ref):
        r = pl.program_id(1)
        w = w_ref[pl.ds(r * D, D), :]
        b = b_ref[0, pl.ds(r * D, D)]
        o_ref[...] = lax.dot_general(
            x_ref[...], w, (((1,), (1,)), ((), ())),
            preferred_element_type=jnp.float32) + b
    return pl.pallas_call(
        body,
        grid=(nb, R),
        in_specs=[
            pl.BlockSpec((BN, D), lambda i, r: (i, 0)),
            pl.BlockSpec((D * R, D), lambda i, r: (0, 0)),
            pl.BlockSpec((1, D * R), lambda i, r: (0, 0)),
        ],
        out_specs=pl.BlockSpec((BN, D), lambda i, r: (r * nb + i, 0)),
        out_shape=jax.ShapeDtypeStruct((R * N, D), jnp.float32),
    )(x, Wl, b2)


def _transform2(p, Wl, b2):
    nb = N // BN
    nbh = nb // NC

    def body(p_ref, w_ref, b_ref, o_ref):
        r = pl.program_id(1)
        w = w_ref[pl.ds(r * D, D), :]
        b = b_ref[0, pl.ds(r * D, D)]
        x = jnp.maximum(p_ref[0], 0.0)
        o_ref[...] = lax.dot_general(
            x, w, (((1,), (1,)), ((), ())),
            preferred_element_type=jnp.float32) + b
    return pl.pallas_call(
        body,
        grid=(nb, R),
        in_specs=[
            pl.BlockSpec((1, BN, D), lambda i, r: (i // nbh, i % nbh, 0)),
            pl.BlockSpec((D * R, D), lambda i, r: (0, 0)),
            pl.BlockSpec((1, D * R), lambda i, r: (0, 0)),
        ],
        out_specs=pl.BlockSpec((BN, D), lambda i, r: (r * nb + i, 0)),
        out_shape=jax.ShapeDtypeStruct((R * N, D), jnp.float32),
    )(p, Wl, b2)


def _finalize(p):
    nb = N // BN // NC

    def body(p_ref, g_ref, n_ref):
        i = pl.program_id(0)
        x = jnp.maximum(p_ref[0], 0.0)
        n_ref[...] = x

        @pl.when(i == 0)
        def _():
            g_ref[...] = jnp.zeros_like(g_ref)

        g_ref[...] += jnp.sum(x, axis=0, keepdims=True)

    graph, node = pl.pallas_call(
        body,
        grid=(N // BN,),
        in_specs=[pl.BlockSpec((1, BN, D), lambda i: (i // nb, i % nb, 0))],
        out_specs=[pl.BlockSpec((1, D), lambda i: (0, 0)),
                   pl.BlockSpec((BN, D), lambda i: (i, 0))],
        out_shape=[jax.ShapeDtypeStruct((1, D), jnp.float32),
                   jax.ShapeDtypeStruct((N, D), jnp.float32)],
    )(p)
    return graph.reshape(D), node


def _edge_prep(src, rel, dst):
    mesh = plsc.VectorSubcoreMesh(core_axis_name="c", subcore_axis_name="s")

    @functools.partial(
        pl.kernel,
        mesh=mesh,
        compiler_params=pltpu.CompilerParams(needs_layout_passes=False),
        out_type=(
            jax.ShapeDtypeStruct((NC * NS * CAP,), jnp.int32),
            jax.ShapeDtypeStruct((NC * NS * CAP,), jnp.int32),
            jax.ShapeDtypeStruct((NC * NS * 16,), jnp.int32),
        ),
        scratch_types=[
            pltpu.VMEM((SCH,), jnp.int32),
            pltpu.VMEM((SCH,), jnp.int32),
            pltpu.VMEM((SCH,), jnp.int32),
            pltpu.VMEM((SCH,), jnp.int32),
            pltpu.VMEM((SCH,), jnp.int32),
            pltpu.VMEM((SCH,), jnp.int32),
            pltpu.VMEM((CAP,), jnp.int32),
            pltpu.VMEM((CAP,), jnp.int32),
            pltpu.VMEM((16,), jnp.int32),
            pltpu.SemaphoreType.DMA,
            pltpu.SemaphoreType.DMA,
        ],
    )
    def body(src_h, rel_h, dst_h, cg_h, cd_h, cnt_h,
             src_s0, src_s1, rel_s0, rel_s1, dst_s0, dst_s1,
             ccg, ccd, cbuf, sem0, sem1):
        src_s = (src_s0, src_s1)
        rel_s = (rel_s0, rel_s1)
        dst_s = (dst_s0, dst_s1)
        cid = lax.axis_index("c")
        sid = lax.axis_index("s")
        sems = (sem0, sem1)
        e0 = sid * EPS
        lo = cid * HALF

        def fire(m, b):
            base = pl.multiple_of(e0 + m * SCH, 8)
            pltpu.async_copy(src_h.at[pl.ds(base, SCH)], src_s[b], sems[b])
            pltpu.async_copy(rel_h.at[pl.ds(base, SCH)], rel_s[b], sems[b])
            pltpu.async_copy(dst_h.at[pl.ds(base, SCH)], dst_s[b], sems[b])

        def wait(b):
            for ref in (src_s, rel_s, dst_s):
                pltpu.make_async_copy(src_h.at[pl.ds(0, SCH)], ref[b],
                                      sems[b]).wait()

        fire(0, 0)
        iota = lax.iota(jnp.int32, 16)

        def pre(i, c):
            sl = pl.ds(i * 16, 16)
            ccg[sl] = (iota + i * 16) & 8191
            ccd[sl] = HALF + ((iota + i) & 63)
            return c
        lax.fori_loop(0, CAP // 16, pre, 0)

        def mega(i, fillv):
            for b in range(2):
                m = i * 2 + b
                wait(b)

                @pl.when(m + 1 < NMEGA)
                def _():
                    fire(m + 1, 1 - b)

                def scan(v, fv):
                    sl = pl.ds(v * 16, 16)
                    s = src_s[b][sl]
                    r = rel_s[b][sl]
                    d = dst_s[b][sl] - lo
                    g = r * N + s
                    keep = (d >= 0) & (d < HALF)
                    ki = keep.astype(jnp.int32)
                    pos = fv * 16 + iota
                    plsc.store_scatter(ccg, [pos], g, mask=keep)
                    plsc.store_scatter(ccd, [pos], d, mask=keep)
                    return fv + ki
                fillv = lax.fori_loop(0, SCH // 16, scan, fillv)
            return fillv
        fillv = lax.fori_loop(0, NMEGA // 2, mega, jnp.zeros(16, jnp.int32))

        mx = fillv[0]
        for l in range(1, 16):
            mx = jnp.maximum(mx, fillv[l])
        nchunks = (mx * 16 + CH - 1) // CH

        obase = (cid * NS + sid) * CAP
        pltpu.sync_copy(ccg, cg_h.at[pl.ds(obase, CAP)])
        pltpu.sync_copy(ccd, cd_h.at[pl.ds(obase, CAP)])
        cbuf[...] = jnp.full((16,), nchunks, jnp.int32)
        pltpu.sync_copy(cbuf, cnt_h.at[pl.ds((cid * NS + sid) * 16, 16)])

    return body(src, rel, dst)


def _edge_scatter(cg, cd, cnt, z, zero, final=False):
    mesh = plsc.VectorSubcoreMesh(core_axis_name="c", subcore_axis_name="s")

    if final:
        out_type = (jax.ShapeDtypeStruct((N, D), jnp.float32),
                    jax.ShapeDtypeStruct((NC * NS * D,), jnp.float32))
    else:
        out_type = jax.ShapeDtypeStruct((NC, NPH, D), jnp.float32)

    @functools.partial(
        pl.kernel,
        mesh=mesh,
        compiler_params=pltpu.CompilerParams(needs_layout_passes=False),
        out_type=out_type,
        scratch_types=[
            pltpu.VMEM((NIB, CH), jnp.int32),
            pltpu.VMEM((NIB, CH), jnp.int32),
            pltpu.VMEM((NRB, CH, D), jnp.float32),
            pltpu.VMEM((16,), jnp.int32),
            pltpu.VMEM((D,), jnp.float32),
            pltpu.VMEM_SHARED((NPH, D), jnp.float32),
        ] + [pltpu.SemaphoreType.DMA] * (NIB + 2 * NRB + 1),
    )
    def body(cg_h, cd_h, cnt_h, z_h, zero_h, *rest):
        if final:
            node_h, gp_h = rest[0], rest[1]
            gbuf, dbuf, rows_v, cbuf, cbuf2, acc = rest[2:8]
            sems = rest[8:]
        else:
            out_h = rest[0]
            gbuf, dbuf, rows_v, cbuf, cbuf2, acc = rest[1:7]
            sems = rest[7:]
        sem_i = sems[:NIB]
        sem_g = sems[NIB:NIB + NRB]
        sem_s = sems[NIB + NRB:NIB + 2 * NRB]
        sem_c = sems[NIB + 2 * NRB]
        cid = lax.axis_index("c")
        sid = lax.axis_index("s")
        rpt = NPH // NS
        r0 = sid * rpt
        obase = (cid * NS + sid) * CAP

        pltpu.async_copy(cnt_h.at[pl.ds((cid * NS + sid) * 16, 16)], cbuf,
                         sem_c)
        pltpu.sync_copy(zero_h.at[pl.ds(r0, rpt)], acc.at[pl.ds(r0, rpt)])
        pltpu.make_async_copy(cnt_h.at[pl.ds(0, 16)], cbuf, sem_c).wait()
        nchunks = cbuf[pl.ds(0, 16)][0]
        plsc.subcore_barrier()

        def fire_idx(k, ib):
            base = pl.multiple_of(obase + k * CH, 8)
            pltpu.async_copy(cg_h.at[pl.ds(base, CH)], gbuf.at[ib], sem_i[ib])
            pltpu.async_copy(cd_h.at[pl.ds(base, CH)], dbuf.at[ib], sem_i[ib])

        def wait_idx(ib):
            pltpu.make_async_copy(cg_h.at[pl.ds(0, CH)], gbuf.at[ib],
                                  sem_i[ib]).wait()
            pltpu.make_async_copy(cg_h.at[pl.ds(0, CH)], dbuf.at[ib],
                                  sem_i[ib]).wait()

        def fire_gather(ib, rb):
            pltpu.async_copy(z_h.at[gbuf.at[ib]], rows_v.at[rb], sem_g[rb])

        def wait_gather(rb):
            pltpu.make_async_copy(z_h.at[pl.ds(0, CH)], rows_v.at[rb],
                                  sem_g[rb]).wait()

        def fire_scatter(rb, ib):
            pltpu.async_copy(rows_v.at[rb], acc.at[dbuf.at[ib]], sem_s[rb],
                             add=True)

        def wait_scatter(rb):
            pltpu.make_async_copy(rows_v.at[rb], acc.at[pl.ds(0, CH)],
                                  sem_s[rb]).wait()

        for u in range(3):
            @pl.when(u < nchunks)
            def _():
                fire_idx(u, u)

        @pl.when(0 < nchunks)
        def _():
            wait_idx(0)
            fire_gather(0, 0)

        def batch(i, c):
            for u in range(4):
                k = i * 4 + u
                rb = u % 2
                ib = u

                @pl.when((k >= 1) & (k - 1 < nchunks))
                def _():
                    wait_scatter((u + 1) % 2)

                @pl.when(k + 3 < nchunks)
                def _():
                    fire_idx(k + 3, (u + 3) % 4)

                @pl.when(k + 1 < nchunks)
                def _():
                    wait_idx((u + 1) % 4)
                    fire_gather((u + 1) % 4, (u + 1) % 2)

                @pl.when(k < nchunks)
                def _():
                    wait_gather(rb)
                    fire_scatter(rb, ib)
            return c
        lax.fori_loop(0, (nchunks + 4) // 4, batch, 0)

        plsc.subcore_barrier()
        if not final:
            pltpu.sync_copy(acc.at[pl.ds(r0, rpt)],
                            out_h.at[cid, pl.ds(r0, rpt)])
            return

        FR = 40
        gz = [jnp.zeros(16, jnp.float32) for _ in range(D // 16)]

        def fchunk(j, gs):
            lbase = r0 + j * FR

            def work():
                pltpu.sync_copy(acc.at[pl.ds(lbase, FR)],
                                rows_v.at[0, pl.ds(0, FR)])
                gs2 = list(gs)

                def rrow(t, carry):
                    carry2 = list(carry)
                    for q in range(D // 16):
                        sl = pl.ds(q * 16, 16)
                        v = jnp.maximum(rows_v[0, t, sl], 0.0)
                        rows_v[0, t, sl] = v
                        carry2[q] = carry2[q] + v
                    return tuple(carry2)
                gs2 = lax.fori_loop(0, FR, rrow, tuple(gs2))
                pltpu.sync_copy(
                    rows_v.at[0, pl.ds(0, FR)],
                    node_h.at[pl.ds(cid * HALF + lbase, FR)])
                return gs2
            return lax.cond(lbase < HALF, work, lambda: gs)
        gz = lax.fori_loop(0, rpt // FR, fchunk, tuple(gz))

        for q in range(D // 16):
            cbuf2[pl.ds(q * 16, 16)] = gz[q]
        pltpu.sync_copy(cbuf2,
                        gp_h.at[pl.ds((cid * NS + sid) * D, D)])

    return body(cg, cd, cnt, z, zero)


def kernel(input, edge_index, edge_relation, Ws0, bs0, Wl0, bl0,
           Ws1, bs1, Wl1, bl1):
    x = input
    src = edge_index[0]
    dst = edge_index[1]
    rel = edge_relation
    zero = jnp.zeros((NPH, D), jnp.float32)

    cg, cd, cnt = _edge_prep(src, rel, dst)
    z1 = _transform1(x, Wl0, bl0.reshape(1, D * R))
    p1 = _edge_scatter(cg, cd, cnt, z1, zero)
    z2 = _transform2(p1, Wl1, bl1.reshape(1, D * R))
    node, gpart = _edge_scatter(cg, cd, cnt, z2, zero, final=True)
    graph = jnp.sum(gpart.reshape(NC * NS, D), axis=0)
    return graph, node


# speedup vs baseline: 18.5590x; 1.0343x over previous
import functools

import jax
import jax.numpy as jnp
from jax import lax
from jax.experimental import pallas as pl
from jax.experimental.pallas import tpu as pltpu
from jax.experimental.pallas import tpu_sc as plsc

N = 10000
E = 320000
D = 128
R = 7

NC = 2
NS = 16
HALF = N // 2
NPH = 5120
EPS = 20480
SCH = 2560
NMEGA = 8
CAP = 20592
CH = 88
CAPCH = CAP // CH
NIB = 4
NRB = 2
NBATCH = 58

BN = 5000


def _transform1(x, Wl, b2):
    nb = N // BN

    def body(x_ref, w_ref, b_ref, o_ref):
        r = pl.program_id(1)
        w = w_ref[pl.ds(r * D, D), :]
        b = b_ref[0, pl.ds(r * D, D)]
        o_ref[...] = lax.dot_general(
            x_ref[...], w, (((1,), (1,)), ((), ())),
            preferred_element_type=jnp.float32) + b
    return pl.pallas_call(
        body,
        grid=(nb, R),
        in_specs=[
            pl.BlockSpec((BN, D), lambda i, r: (i, 0)),
            pl.BlockSpec((D * R, D), lambda i, r: (0, 0)),
            pl.BlockSpec((1, D * R), lambda i, r: (0, 0)),
        ],
        out_specs=pl.BlockSpec((BN, D), lambda i, r: (r * nb + i, 0)),
        out_shape=jax.ShapeDtypeStruct((R * N, D), jnp.float32),
    )(x, Wl, b2)


def _transform2(p, Wl, b2):
    nb = N // BN
    nbh = nb // NC

    def body(p_ref, w_ref, b_ref, o_ref):
        r = pl.program_id(1)
        w = w_ref[pl.ds(r * D, D), :]
        b = b_ref[0, pl.ds(r * D, D)]
        x = jnp.maximum(p_ref[0], 0.0)
        o_ref[...] = lax.dot_general(
            x, w, (((1,), (1,)), ((), ())),
            preferred_element_type=jnp.float32) + b
    return pl.pallas_call(
        body,
        grid=(nb, R),
        in_specs=[
            pl.BlockSpec((1, BN, D), lambda i, r: (i // nbh, i % nbh, 0)),
            pl.BlockSpec((D * R, D), lambda i, r: (0, 0)),
            pl.BlockSpec((1, D * R), lambda i, r: (0, 0)),
        ],
        out_specs=pl.BlockSpec((BN, D), lambda i, r: (r * nb + i, 0)),
        out_shape=jax.ShapeDtypeStruct((R * N, D), jnp.float32),
    )(p, Wl, b2)


def _finalize(p):
    nb = N // BN // NC

    def body(p_ref, g_ref, n_ref):
        i = pl.program_id(0)
        x = jnp.maximum(p_ref[0], 0.0)
        n_ref[...] = x

        @pl.when(i == 0)
        def _():
            g_ref[...] = jnp.zeros_like(g_ref)

        g_ref[...] += jnp.sum(x, axis=0, keepdims=True)

    graph, node = pl.pallas_call(
        body,
        grid=(N // BN,),
        in_specs=[pl.BlockSpec((1, BN, D), lambda i: (i // nb, i % nb, 0))],
        out_specs=[pl.BlockSpec((1, D), lambda i: (0, 0)),
                   pl.BlockSpec((BN, D), lambda i: (i, 0))],
        out_shape=[jax.ShapeDtypeStruct((1, D), jnp.float32),
                   jax.ShapeDtypeStruct((N, D), jnp.float32)],
    )(p)
    return graph.reshape(D), node


def _edge_prep(ei, rel):
    mesh = plsc.VectorSubcoreMesh(core_axis_name="c", subcore_axis_name="s")

    @functools.partial(
        pl.kernel,
        mesh=mesh,
        compiler_params=pltpu.CompilerParams(needs_layout_passes=False),
        out_type=(
            jax.ShapeDtypeStruct((NC * NS * CAP,), jnp.int32),
            jax.ShapeDtypeStruct((NC * NS * CAP,), jnp.int32),
            jax.ShapeDtypeStruct((NC * NS * 16,), jnp.int32),
        ),
        scratch_types=[
            pltpu.VMEM((2, SCH), jnp.int32),
            pltpu.VMEM((2, SCH), jnp.int32),
            pltpu.VMEM((SCH,), jnp.int32),
            pltpu.VMEM((SCH,), jnp.int32),
            pltpu.VMEM((CAP,), jnp.int32),
            pltpu.VMEM((CAP,), jnp.int32),
            pltpu.VMEM((16,), jnp.int32),
            pltpu.SemaphoreType.DMA,
            pltpu.SemaphoreType.DMA,
        ],
    )
    def body(ei_h, rel_h, cg_h, cd_h, cnt_h,
             ei_s0, ei_s1, rel_s0, rel_s1,
             ccg, ccd, cbuf, sem0, sem1):
        ei_s = (ei_s0, ei_s1)
        rel_s = (rel_s0, rel_s1)
        cid = lax.axis_index("c")
        sid = lax.axis_index("s")
        sems = (sem0, sem1)
        e0 = sid * EPS
        lo = cid * HALF
        nm = jnp.where(sid == NS - 1, 5, NMEGA)

        def fire(m, b):
            base = pl.multiple_of(e0 + m * SCH, 128)
            pltpu.async_copy(ei_h.at[:, pl.ds(base, SCH)], ei_s[b], sems[b])
            pltpu.async_copy(rel_h.at[pl.ds(base, SCH)], rel_s[b], sems[b])

        def wait(b):
            pltpu.make_async_copy(ei_h.at[:, pl.ds(0, SCH)], ei_s[b],
                                  sems[b]).wait()
            pltpu.make_async_copy(rel_h.at[pl.ds(0, SCH)], rel_s[b],
                                  sems[b]).wait()

        fire(0, 0)
        iota = lax.iota(jnp.int32, 16)

        def pre(i, c):
            sl = pl.ds(i * 16, 16)
            ccg[sl] = (iota + i * 16) & 8191
            ccd[sl] = HALF + ((iota + i) & 63)
            return c
        lax.fori_loop(0, CAP // 16, pre, 0)

        def mega(i, fillv):
            for b in range(2):
                m = i * 2 + b

                @pl.when(m + 1 < nm)
                def _():
                    fire(m + 1, 1 - b)

                def scan(v, fv):
                    sl = pl.ds(v * 16, 16)
                    s = ei_s[b][0, sl]
                    r = rel_s[b][sl]
                    d = ei_s[b][1, sl] - lo
                    g = r * N + s
                    keep = (d >= 0) & (d < HALF)
                    ki = keep.astype(jnp.int32)
                    pos = fv * 16 + iota
                    plsc.store_scatter(ccg, [pos], g, mask=keep)
                    plsc.store_scatter(ccd, [pos], d, mask=keep)
                    return fv + ki

                def active(fv):
                    wait(b)
                    return lax.fori_loop(0, SCH // 16, scan, fv)
                fillv = lax.cond(m < nm, active, lambda fv: fv, fillv)
            return fillv
        fillv = lax.fori_loop(0, NMEGA // 2, mega, jnp.zeros(16, jnp.int32))

        mx = fillv[0]
        for l in range(1, 16):
            mx = jnp.maximum(mx, fillv[l])
        nchunks = (mx * 16 + CH - 1) // CH

        obase = (cid * NS + sid) * CAP
        pltpu.sync_copy(ccg, cg_h.at[pl.ds(obase, CAP)])
        pltpu.sync_copy(ccd, cd_h.at[pl.ds(obase, CAP)])
        cbuf[...] = jnp.full((16,), nchunks, jnp.int32)
        pltpu.sync_copy(cbuf, cnt_h.at[pl.ds((cid * NS + sid) * 16, 16)])

    return body(ei, rel)


def _edge_scatter(cg, cd, cnt, z, zero, final=False):
    mesh = plsc.VectorSubcoreMesh(core_axis_name="c", subcore_axis_name="s")

    if final:
        out_type = (jax.ShapeDtypeStruct((N, D), jnp.float32),
                    jax.ShapeDtypeStruct((NC * NS * D,), jnp.float32))
    else:
        out_type = jax.ShapeDtypeStruct((NC, NPH, D), jnp.float32)

    @functools.partial(
        pl.kernel,
        mesh=mesh,
        compiler_params=pltpu.CompilerParams(needs_layout_passes=False),
        out_type=out_type,
        scratch_types=[
            pltpu.VMEM((NIB, CH), jnp.int32),
            pltpu.VMEM((NIB, CH), jnp.int32),
            pltpu.VMEM((NRB, CH, D), jnp.float32),
            pltpu.VMEM((16,), jnp.int32),
            pltpu.VMEM((D,), jnp.float32),
            pltpu.VMEM_SHARED((NPH, D), jnp.float32),
        ] + [pltpu.SemaphoreType.DMA] * (NIB + 2 * NRB + 1),
    )
    def body(cg_h, cd_h, cnt_h, z_h, zero_h, *rest):
        if final:
            node_h, gp_h = rest[0], rest[1]
            gbuf, dbuf, rows_v, cbuf, cbuf2, acc = rest[2:8]
            sems = rest[8:]
        else:
            out_h = rest[0]
            gbuf, dbuf, rows_v, cbuf, cbuf2, acc = rest[1:7]
            sems = rest[7:]
        sem_i = sems[:NIB]
        sem_g = sems[NIB:NIB + NRB]
        sem_s = sems[NIB + NRB:NIB + 2 * NRB]
        sem_c = sems[NIB + 2 * NRB]
        cid = lax.axis_index("c")
        sid = lax.axis_index("s")
        rpt = NPH // NS
        r0 = sid * rpt
        obase = (cid * NS + sid) * CAP

        pltpu.async_copy(cnt_h.at[pl.ds((cid * NS + sid) * 16, 16)], cbuf,
                         sem_c)
        pltpu.sync_copy(zero_h.at[pl.ds(r0, rpt)], acc.at[pl.ds(r0, rpt)])
        pltpu.make_async_copy(cnt_h.at[pl.ds(0, 16)], cbuf, sem_c).wait()
        nchunks = cbuf[pl.ds(0, 16)][0]
        plsc.subcore_barrier()

        def fire_idx(k, ib):
            base = pl.multiple_of(obase + k * CH, 8)
            pltpu.async_copy(cg_h.at[pl.ds(base, CH)], gbuf.at[ib], sem_i[ib])
            pltpu.async_copy(cd_h.at[pl.ds(base, CH)], dbuf.at[ib], sem_i[ib])

        def wait_idx(ib):
            pltpu.make_async_copy(cg_h.at[pl.ds(0, CH)], gbuf.at[ib],
                                  sem_i[ib]).wait()
            pltpu.make_async_copy(cg_h.at[pl.ds(0, CH)], dbuf.at[ib],
                                  sem_i[ib]).wait()

        def fire_gather(ib, rb):
            pltpu.async_copy(z_h.at[gbuf.at[ib]], rows_v.at[rb], sem_g[rb])

        def wait_gather(rb):
            pltpu.make_async_copy(z_h.at[pl.ds(0, CH)], rows_v.at[rb],
                                  sem_g[rb]).wait()

        def fire_scatter(rb, ib):
            pltpu.async_copy(rows_v.at[rb], acc.at[dbuf.at[ib]], sem_s[rb],
                             add=True)

        def wait_scatter(rb):
            pltpu.make_async_copy(rows_v.at[rb], acc.at[pl.ds(0, CH)],
                                  sem_s[rb]).wait()

        for u in range(3):
            @pl.when(u < nchunks)
            def _():
                fire_idx(u, u)

        @pl.when(0 < nchunks)
        def _():
            wait_idx(0)
            fire_gather(0, 0)

        def batch(i, c):
            for u in range(4):
                k = i * 4 + u
                rb = u % 2
                ib = u

                @pl.when((k >= 1) & (k - 1 < nchunks))
                def _():
                    wait_scatter((u + 1) % 2)

                @pl.when(k + 3 < nchunks)
                def _():
                    fire_idx(k + 3, (u + 3) % 4)

                @pl.when(k + 1 < nchunks)
                def _():
                    wait_idx((u + 1) % 4)
                    fire_gather((u + 1) % 4, (u + 1) % 2)

                @pl.when(k < nchunks)
                def _():
                    wait_gather(rb)
                    fire_scatter(rb, ib)
            return c
        lax.fori_loop(0, (nchunks + 4) // 4, batch, 0)

        plsc.subcore_barrier()
        if not final:
            pltpu.sync_copy(acc.at[pl.ds(r0, rpt)],
                            out_h.at[cid, pl.ds(r0, rpt)])
            return

        FR = 40
        gz = [jnp.zeros(16, jnp.float32) for _ in range(D // 16)]

        def fchunk(j, gs):
            lbase = r0 + j * FR

            def work():
                pltpu.sync_copy(acc.at[pl.ds(lbase, FR)],
                                rows_v.at[0, pl.ds(0, FR)])
                gs2 = list(gs)

                def rrow(t, carry):
                    carry2 = list(carry)
                    for q in range(D // 16):
                        sl = pl.ds(q * 16, 16)
                        v = jnp.maximum(rows_v[0, t, sl], 0.0)
                        rows_v[0, t, sl] = v
                        carry2[q] = carry2[q] + v
                    return tuple(carry2)
                gs2 = lax.fori_loop(0, FR, rrow, tuple(gs2))
                pltpu.sync_copy(
                    rows_v.at[0, pl.ds(0, FR)],
                    node_h.at[pl.ds(cid * HALF + lbase, FR)])
                return gs2
            return lax.cond(lbase < HALF, work, lambda: gs)
        gz = lax.fori_loop(0, rpt // FR, fchunk, tuple(gz))

        for q in range(D // 16):
            cbuf2[pl.ds(q * 16, 16)] = gz[q]
        pltpu.sync_copy(cbuf2,
                        gp_h.at[pl.ds((cid * NS + sid) * D, D)])

    return body(cg, cd, cnt, z, zero)


def kernel(input, edge_index, edge_relation, Ws0, bs0, Wl0, bl0,
           Ws1, bs1, Wl1, bl1):
    x = input
    zero = jnp.zeros((NPH, D), jnp.float32)

    cg, cd, cnt = _edge_prep(edge_index, edge_relation)
    z1 = _transform1(x, Wl0, bl0.reshape(1, D * R))
    p1 = _edge_scatter(cg, cd, cnt, z1, zero)
    z2 = _transform2(p1, Wl1, bl1.reshape(1, D * R))
    node, gpart = _edge_scatter(cg, cd, cnt, z2, zero, final=True)
    graph = jnp.sum(gpart.reshape(NC * NS, D), axis=0)
    return graph, node
